# baseline (speedup 1.0000x reference)
import functools

import jax
import jax.numpy as jnp
from jax import lax
from jax.experimental import pallas as pl
from jax.experimental.pallas import tpu as pltpu
from jax.experimental.pallas import tpu_sc as plsc

N = 10000
NPAD = 10240
E = 160000
EPAD = 163840
DIN = 128
DH = 512
DOUT = 128
G = 64

NC = 2
NS = 16
W = 128
RPT = NPAD // NS
C1 = DIN + 16

_mesh = plsc.VectorSubcoreMesh(core_axis_name="c", subcore_axis_name="s")


def _edge_loop_db(h_hbm, idxs, wbase, rows_a, rows_b, acc, gsa, gsb, nwin,
                  dget):
    pltpu.async_copy(h_hbm.at[idxs.at[wbase]], rows_a, gsa)

    def body(k, carry):
        j0 = 2 * k
        pltpu.make_async_copy(
            h_hbm.at[idxs.at[wbase + j0]], rows_a, gsa).wait()
        db = pltpu.async_copy(h_hbm.at[idxs.at[wbase + j0 + 1]], rows_b, gsb)
        pltpu.sync_copy(rows_a, acc.at[dget(j0)], add=True)
        db.wait()

        @pl.when(j0 + 2 < nwin)
        def _issue_a():
            pltpu.async_copy(h_hbm.at[idxs.at[wbase + j0 + 2]], rows_a, gsa)

        pltpu.sync_copy(rows_b, acc.at[dget(j0 + 1)], add=True)
        return carry

    lax.fori_loop(0, nwin // 2, body, 0)


def _make_edge_split_agg(C, WL):
    EPW = EPAD // (NC * NS)
    NWIN = EPW // WL

    @functools.partial(
        pl.kernel,
        out_type=jax.ShapeDtypeStruct((NC * NPAD, C), jnp.float32),
        mesh=_mesh,
        scratch_types=[
            pltpu.VMEM((NWIN, WL), jnp.int32),
            pltpu.VMEM((NWIN, WL), jnp.int32),
            pltpu.VMEM((WL, C), jnp.float32),
            pltpu.VMEM((WL, C), jnp.float32),
            pltpu.VMEM_SHARED((NPAD, C), jnp.float32),
            pltpu.SemaphoreType.DMA,
            pltpu.SemaphoreType.DMA,
        ],
        compiler_params=pltpu.CompilerParams(use_tc_tiling_on_sc=False),
    )
    def agg(h_hbm, src_hbm, dst_hbm, zer_hbm, out_hbm, idxs, idxd,
            rows_a, rows_b, acc, gsa, gsb):
        c = lax.axis_index("c")
        s = lax.axis_index("s")
        w = s * NC + c
        r0 = s * RPT
        pltpu.sync_copy(zer_hbm.at[pl.ds(r0, RPT)], acc.at[pl.ds(r0, RPT)])
        pltpu.sync_copy(src_hbm.at[pl.ds(w * NWIN, NWIN)], idxs)
        pltpu.sync_copy(dst_hbm.at[pl.ds(w * NWIN, NWIN)], idxd)
        plsc.subcore_barrier()
        _edge_loop_db(h_hbm, idxs, 0, rows_a, rows_b, acc, gsa, gsb,
                      NWIN, lambda j: idxd.at[j])
        plsc.subcore_barrier()
        pltpu.sync_copy(acc.at[pl.ds(r0, RPT)],
                        out_hbm.at[pl.ds(c * NPAD + r0, RPT)])

    return agg


WL1 = 64
_agg_l1 = _make_edge_split_agg(C1, WL1)
_agg_l3 = _make_edge_split_agg(DOUT, W)


def _make_chunk_agg():
    EPT = EPAD // NS
    NWIN = EPT // W

    NHALF = NWIN // 2

    @functools.partial(
        pl.kernel,
        out_type=[jax.ShapeDtypeStruct((NPAD, DIN), jnp.float32)] * 4,
        mesh=_mesh,
        scratch_types=[
            pltpu.VMEM((NWIN, W), jnp.int32),
            pltpu.VMEM((NHALF, W), jnp.int32),
            pltpu.VMEM((W, DIN), jnp.float32),
            pltpu.VMEM((W, DIN), jnp.float32),
            pltpu.VMEM_SHARED((NPAD, DIN), jnp.float32),
            pltpu.SemaphoreType.DMA,
            pltpu.SemaphoreType.DMA,
        ],
        compiler_params=pltpu.CompilerParams(use_tc_tiling_on_sc=False),
    )
    def agg4(h0, h1, h2, h3, src_hbm, dst_hbm, zer_hbm,
             o0, o1, o2, o3, idxs, idxd, rows_a, rows_b, acc, gsa, gsb):
        c = lax.axis_index("c")
        s = lax.axis_index("s")
        r0 = s * RPT
        hs = (h0, h1, h2, h3)
        os_ = (o0, o1, o2, o3)
        pltpu.sync_copy(src_hbm.at[pl.ds(s * NWIN, NWIN)], idxs)

        for chunk in range(4):
            h_hbm = hs[chunk]
            out_hbm = os_[chunk]

            @pl.when(c == (chunk % 2))
            def _process():
                pltpu.sync_copy(zer_hbm.at[pl.ds(r0, RPT)],
                                acc.at[pl.ds(r0, RPT)])
                plsc.subcore_barrier()
                for half in range(2):
                    pltpu.sync_copy(
                        dst_hbm.at[pl.ds(s * NWIN + half * NHALF, NHALF)],
                        idxd)
                    _edge_loop_db(h_hbm, idxs, half * NHALF, rows_a, rows_b,
                                  acc, gsa, gsb, NHALF, lambda j: idxd.at[j])
                plsc.subcore_barrier()
                pltpu.sync_copy(acc.at[pl.ds(r0, RPT)],
                                out_hbm.at[pl.ds(r0, RPT)])
                plsc.subcore_barrier()

    return agg4


_agg_l2 = _make_chunk_agg()

R = 256
NBLK = NPAD // R


def _l1_body(s1_ref, x_ref, wl_ref, bl_ref, wr_ref,
             h0_ref, h1_ref, h2_ref, h3_ref, rb_ref):
    ssum = s1_ref[0] + s1_ref[1]
    cnt = ssum[:, DIN:DIN + 1]
    recip = 1.0 / jnp.maximum(cnt, 1.0)
    aggv = ssum[:, :DIN] * recip
    h = (jnp.dot(aggv, wl_ref[...], preferred_element_type=jnp.float32)
         + bl_ref[...]
         + jnp.dot(x_ref[...], wr_ref[...], preferred_element_type=jnp.float32))
    h = jnp.maximum(h, 0.0)
    h0_ref[...] = h[:, 0:128]
    h1_ref[...] = h[:, 128:256]
    h2_ref[...] = h[:, 256:384]
    h3_ref[...] = h[:, 384:512]
    rb_ref[...] = jnp.broadcast_to(recip, (R, DIN))


def _tc_layer1(s1, x_pad, wl1, bl1, wr1):
    blk = lambda i: (i, 0)
    whole = lambda i: (0, 0)
    outs = jax.ShapeDtypeStruct((NPAD, DIN), jnp.float32)
    return pl.pallas_call(
        _l1_body,
        grid=(NBLK,),
        in_specs=[
            pl.BlockSpec((2, R, C1), lambda i: (0, i, 0)),
            pl.BlockSpec((R, DIN), blk),
            pl.BlockSpec((DIN, DH), whole),
            pl.BlockSpec((1, DH), whole),
            pl.BlockSpec((DIN, DH), whole),
        ],
        out_specs=[pl.BlockSpec((R, DIN), blk)] * 5,
        out_shape=[outs] * 5,
    )(s1, x_pad, wl1, bl1, wr1)


def _l2_body(s20, s21, s22, s23, h10, h11, h12, h13, rb_ref,
             wl2_ref, bl2_ref, wr2_ref, wl3_ref, wr3_ref,
             p3_ref, r3_ref):
    recip = rb_ref[:, 0:1]
    aggv = jnp.concatenate(
        [s20[...], s21[...], s22[...], s23[...]], axis=1) * recip
    h1 = jnp.concatenate([h10[...], h11[...], h12[...], h13[...]], axis=1)
    h2 = (jnp.dot(aggv, wl2_ref[...], preferred_element_type=jnp.float32)
          + bl2_ref[...]
          + jnp.dot(h1, wr2_ref[...], preferred_element_type=jnp.float32))
    h2 = jnp.maximum(h2, 0.0)
    p3_ref[...] = jnp.dot(h2, wl3_ref[...], preferred_element_type=jnp.float32)
    r3_ref[...] = jnp.dot(h2, wr3_ref[...], preferred_element_type=jnp.float32)


def _tc_layer2(s2s, h1s, recipb, wl2, bl2, wr2, wl3, wr3):
    blk = lambda i: (i, 0)
    whole = lambda i: (0, 0)
    outs = jax.ShapeDtypeStruct((NPAD, DOUT), jnp.float32)
    return pl.pallas_call(
        _l2_body,
        grid=(NBLK,),
        in_specs=(
            [pl.BlockSpec((R, DIN), blk)] * 8
            + [pl.BlockSpec((R, DIN), blk)]
            + [pl.BlockSpec((DH, DH), whole),
               pl.BlockSpec((1, DH), whole),
               pl.BlockSpec((DH, DH), whole),
               pl.BlockSpec((DH, DOUT), whole),
               pl.BlockSpec((DH, DOUT), whole)]
        ),
        out_specs=[pl.BlockSpec((R, DOUT), blk)] * 2,
        out_shape=[outs] * 2,
    )(*s2s, *h1s, recipb, wl2, bl2, wr2, wl3, wr3)


def _final_body(s3_ref, rb_ref, r3_ref, b_ref, bl3_ref, g_ref, be_ref,
                out_ref, psum, csum):
    i = pl.program_id(0)

    @pl.when(i == 0)
    def _init():
        psum[...] = jnp.zeros((G, DOUT), jnp.float32)
        csum[...] = jnp.zeros((G, 1), jnp.float32)

    ssum = s3_ref[0] + s3_ref[1]
    out3 = ssum * rb_ref[:, 0:1] + r3_ref[...] + bl3_ref[...]
    bb = b_ref[0]
    gids = lax.broadcasted_iota(jnp.int32, (G, R), 0).astype(jnp.float32)
    onehot = jnp.where(gids == bb, 1.0, 0.0)
    psum[...] += jnp.dot(onehot, out3, preferred_element_type=jnp.float32)
    csum[...] += jnp.sum(onehot, axis=1, keepdims=True)

    @pl.when(i == NBLK - 1)
    def _finish():
        pooled = psum[...] / jnp.maximum(csum[...], 1.0)
        mu = jnp.mean(pooled, axis=1, keepdims=True)
        var = jnp.mean((pooled - mu) ** 2, axis=1, keepdims=True)
        out_ref[...] = ((pooled - mu) * lax.rsqrt(var + 1e-5)
                        * g_ref[...] + be_ref[...])


def _tc_final(s3, recipb, r3, batchf, bl3, ln_g, ln_b):
    blk = lambda i: (i, 0)
    whole = lambda i: (0, 0)
    return pl.pallas_call(
        _final_body,
        grid=(NBLK,),
        in_specs=[
            pl.BlockSpec((2, R, DOUT), lambda i: (0, i, 0)),
            pl.BlockSpec((R, DIN), blk),
            pl.BlockSpec((R, DOUT), blk),
            pl.BlockSpec((1, 1, R), lambda i: (i, 0, 0)),
            pl.BlockSpec((1, DOUT), whole),
            pl.BlockSpec((1, DOUT), whole),
            pl.BlockSpec((1, DOUT), whole),
        ],
        out_specs=pl.BlockSpec((G, DOUT), whole),
        out_shape=jax.ShapeDtypeStruct((G, DOUT), jnp.float32),
        scratch_shapes=[
            pltpu.VMEM((G, DOUT), jnp.float32),
            pltpu.VMEM((G, 1), jnp.float32),
        ],
    )(s3, recipb, r3, batchf, bl3, ln_g, ln_b)


def kernel(x, edge_index, batch, Wl1, bl1, Wr1, Wl2, bl2, Wr2,
           Wl3, bl3, Wr3, ln_g, ln_b):
    f32 = jnp.float32
    src = edge_index[0]
    dst = edge_index[1]
    padidx = (N + (jnp.arange(EPAD - E, dtype=jnp.int32) % (NPAD - N)))
    srcf = jnp.concatenate([src, padidx])
    dstf = jnp.concatenate([dst, padidx])
    srcp = srcf.reshape(EPAD // W, W)
    dstp = dstf.reshape(EPAD // W, W)
    srcp64 = srcf.reshape(EPAD // WL1, WL1)
    dstp64 = dstf.reshape(EPAD // WL1, WL1)

    xa = jnp.concatenate(
        [x, jnp.ones((N, 1), f32), jnp.zeros((N, C1 - DIN - 1), f32)], axis=1)
    xa = jnp.concatenate([xa, jnp.zeros((NPAD - N, C1), f32)], axis=0)
    x_pad = jnp.concatenate([x, jnp.zeros((NPAD - N, DIN), f32)], axis=0)

    zer1 = jnp.zeros((NPAD, C1), f32)
    zer = jnp.zeros((NPAD, DIN), f32)

    s1 = _agg_l1(xa, srcp64, dstp64, zer1).reshape(2, NPAD, C1)
    h1s_and_recip = _tc_layer1(s1, x_pad, Wl1, bl1.reshape(1, DH), Wr1)
    h1s, recipb = h1s_and_recip[:4], h1s_and_recip[4]

    s2s = _agg_l2(*h1s, srcp, dstp, zer)
    p3, r3 = _tc_layer2(s2s, h1s, recipb, Wl2, bl2.reshape(1, DH), Wr2,
                        Wl3, Wr3)

    s3 = _agg_l3(p3, srcp, dstp, zer).reshape(2, NPAD, DOUT)
    batchf = jnp.concatenate(
        [batch.astype(f32), jnp.full((NPAD - N,), float(G), f32)]
    ).reshape(NBLK, 1, R)
    out = _tc_final(s3, recipb, r3, batchf, bl3.reshape(1, DOUT),
                    ln_g.reshape(1, DOUT), ln_b.reshape(1, DOUT))
    return out


# speedup vs baseline: 1.0101x; 1.0101x over previous
import functools

import jax
import jax.numpy as jnp
from jax import lax
from jax.experimental import pallas as pl
from jax.experimental.pallas import tpu as pltpu
from jax.experimental.pallas import tpu_sc as plsc

N = 10000
NPAD = 10240
E = 160000
EPAD = 163840
DIN = 128
DH = 512
DOUT = 128
G = 64

NC = 2
NS = 16
W = 128
RPT = NPAD // NS
C1 = DIN + 16

_mesh = plsc.VectorSubcoreMesh(core_axis_name="c", subcore_axis_name="s")


def _edge_loop_db(h_hbm, idxs, wbase, rows_a, rows_b, acc, gsa, gsb, nwin,
                  dget):
    pltpu.async_copy(h_hbm.at[idxs.at[wbase]], rows_a, gsa)

    def body(k, carry):
        j0 = 2 * k
        pltpu.make_async_copy(
            h_hbm.at[idxs.at[wbase + j0]], rows_a, gsa).wait()
        db = pltpu.async_copy(h_hbm.at[idxs.at[wbase + j0 + 1]], rows_b, gsb)
        pltpu.sync_copy(rows_a, acc.at[dget(j0)], add=True)
        db.wait()

        @pl.when(j0 + 2 < nwin)
        def _issue_a():
            pltpu.async_copy(h_hbm.at[idxs.at[wbase + j0 + 2]], rows_a, gsa)

        pltpu.sync_copy(rows_b, acc.at[dget(j0 + 1)], add=True)
        return carry

    lax.fori_loop(0, nwin // 2, body, 0)


def _make_edge_split_agg(C, WL, dt):
    EPW = EPAD // (NC * NS)
    NWIN = EPW // WL

    @functools.partial(
        pl.kernel,
        out_type=jax.ShapeDtypeStruct((NC * NPAD, C), dt),
        mesh=_mesh,
        scratch_types=[
            pltpu.VMEM((NWIN, WL), jnp.int32),
            pltpu.VMEM((NWIN, WL), jnp.int32),
            pltpu.VMEM((WL, C), dt),
            pltpu.VMEM((WL, C), dt),
            pltpu.VMEM_SHARED((NPAD, C), dt),
            pltpu.SemaphoreType.DMA,
            pltpu.SemaphoreType.DMA,
        ],
        compiler_params=pltpu.CompilerParams(use_tc_tiling_on_sc=False),
    )
    def agg(h_hbm, src_hbm, dst_hbm, zer_hbm, out_hbm, idxs, idxd,
            rows_a, rows_b, acc, gsa, gsb):
        c = lax.axis_index("c")
        s = lax.axis_index("s")
        w = s * NC + c
        r0 = s * RPT
        pltpu.sync_copy(zer_hbm.at[pl.ds(r0, RPT)], acc.at[pl.ds(r0, RPT)])
        pltpu.sync_copy(src_hbm.at[pl.ds(w * NWIN, NWIN)], idxs)
        pltpu.sync_copy(dst_hbm.at[pl.ds(w * NWIN, NWIN)], idxd)
        plsc.subcore_barrier()
        _edge_loop_db(h_hbm, idxs, 0, rows_a, rows_b, acc, gsa, gsb,
                      NWIN, lambda j: idxd.at[j])
        plsc.subcore_barrier()
        pltpu.sync_copy(acc.at[pl.ds(r0, RPT)],
                        out_hbm.at[pl.ds(c * NPAD + r0, RPT)])

    return agg


WL1 = 64
BF16 = jnp.bfloat16
_agg_l1 = _make_edge_split_agg(C1, WL1, BF16)
_agg_l3 = _make_edge_split_agg(DOUT, W, BF16)


def _make_chunk_agg():
    EPT = EPAD // NS
    NWIN = EPT // W

    NHALF = NWIN // 2

    @functools.partial(
        pl.kernel,
        out_type=[jax.ShapeDtypeStruct((NPAD, DIN), BF16)] * 4,
        mesh=_mesh,
        scratch_types=[
            pltpu.VMEM((NWIN, W), jnp.int32),
            pltpu.VMEM((NHALF, W), jnp.int32),
            pltpu.VMEM((W, DIN), BF16),
            pltpu.VMEM((W, DIN), BF16),
            pltpu.VMEM_SHARED((NPAD, DIN), BF16),
            pltpu.SemaphoreType.DMA,
            pltpu.SemaphoreType.DMA,
        ],
        compiler_params=pltpu.CompilerParams(use_tc_tiling_on_sc=False),
    )
    def agg4(h0, h1, h2, h3, src_hbm, dst_hbm, zer_hbm,
             o0, o1, o2, o3, idxs, idxd, rows_a, rows_b, acc, gsa, gsb):
        c = lax.axis_index("c")
        s = lax.axis_index("s")
        r0 = s * RPT
        hs = (h0, h1, h2, h3)
        os_ = (o0, o1, o2, o3)
        pltpu.sync_copy(src_hbm.at[pl.ds(s * NWIN, NWIN)], idxs)

        for chunk in range(4):
            h_hbm = hs[chunk]
            out_hbm = os_[chunk]

            @pl.when(c == (chunk % 2))
            def _process():
                pltpu.sync_copy(zer_hbm.at[pl.ds(r0, RPT)],
                                acc.at[pl.ds(r0, RPT)])
                plsc.subcore_barrier()
                for half in range(2):
                    pltpu.sync_copy(
                        dst_hbm.at[pl.ds(s * NWIN + half * NHALF, NHALF)],
                        idxd)
                    _edge_loop_db(h_hbm, idxs, half * NHALF, rows_a, rows_b,
                                  acc, gsa, gsb, NHALF, lambda j: idxd.at[j])
                plsc.subcore_barrier()
                pltpu.sync_copy(acc.at[pl.ds(r0, RPT)],
                                out_hbm.at[pl.ds(r0, RPT)])
                plsc.subcore_barrier()

    return agg4


_agg_l2 = _make_chunk_agg()

R = 256
NBLK = NPAD // R


def _l1_body(s1_ref, x_ref, wl_ref, bl_ref, wr_ref,
             h0_ref, h1_ref, h2_ref, h3_ref, rb_ref):
    ssum = (s1_ref[0].astype(jnp.float32)
            + s1_ref[1].astype(jnp.float32))
    cnt = ssum[:, DIN:DIN + 1]
    recip = 1.0 / jnp.maximum(cnt, 1.0)
    aggv = ssum[:, :DIN] * recip
    h = (jnp.dot(aggv, wl_ref[...], preferred_element_type=jnp.float32)
         + bl_ref[...]
         + jnp.dot(x_ref[...], wr_ref[...], preferred_element_type=jnp.float32))
    h = jnp.maximum(h, 0.0)
    hb = h.astype(jnp.bfloat16)
    h0_ref[...] = hb[:, 0:128]
    h1_ref[...] = hb[:, 128:256]
    h2_ref[...] = hb[:, 256:384]
    h3_ref[...] = hb[:, 384:512]
    rb_ref[...] = jnp.broadcast_to(recip, (R, DIN))


def _tc_layer1(s1, x_pad, wl1, bl1, wr1):
    blk = lambda i: (i, 0)
    whole = lambda i: (0, 0)
    outs_bf = jax.ShapeDtypeStruct((NPAD, DIN), BF16)
    outs_f32 = jax.ShapeDtypeStruct((NPAD, DIN), jnp.float32)
    return pl.pallas_call(
        _l1_body,
        grid=(NBLK,),
        in_specs=[
            pl.BlockSpec((2, R, C1), lambda i: (0, i, 0)),
            pl.BlockSpec((R, DIN), blk),
            pl.BlockSpec((DIN, DH), whole),
            pl.BlockSpec((1, DH), whole),
            pl.BlockSpec((DIN, DH), whole),
        ],
        out_specs=[pl.BlockSpec((R, DIN), blk)] * 5,
        out_shape=[outs_bf] * 4 + [outs_f32],
    )(s1, x_pad, wl1, bl1, wr1)


def _l2_body(s20, s21, s22, s23, h10, h11, h12, h13, rb_ref,
             wl2_ref, bl2_ref, wr2_ref, wl3_ref, wr3_ref,
             p3_ref, r3_ref):
    recip = rb_ref[:, 0:1]
    aggv = jnp.concatenate(
        [s20[...], s21[...], s22[...], s23[...]], axis=1
    ).astype(jnp.float32) * recip
    h1 = jnp.concatenate(
        [h10[...], h11[...], h12[...], h13[...]], axis=1
    ).astype(jnp.float32)
    h2 = (jnp.dot(aggv, wl2_ref[...], preferred_element_type=jnp.float32)
          + bl2_ref[...]
          + jnp.dot(h1, wr2_ref[...], preferred_element_type=jnp.float32))
    h2 = jnp.maximum(h2, 0.0)
    p3_ref[...] = jnp.dot(
        h2, wl3_ref[...], preferred_element_type=jnp.float32
    ).astype(jnp.bfloat16)
    r3_ref[...] = jnp.dot(h2, wr3_ref[...], preferred_element_type=jnp.float32)


def _tc_layer2(s2s, h1s, recipb, wl2, bl2, wr2, wl3, wr3):
    blk = lambda i: (i, 0)
    whole = lambda i: (0, 0)
    outs_bf = jax.ShapeDtypeStruct((NPAD, DOUT), BF16)
    outs_f32 = jax.ShapeDtypeStruct((NPAD, DOUT), jnp.float32)
    return pl.pallas_call(
        _l2_body,
        grid=(NBLK,),
        in_specs=(
            [pl.BlockSpec((R, DIN), blk)] * 8
            + [pl.BlockSpec((R, DIN), blk)]
            + [pl.BlockSpec((DH, DH), whole),
               pl.BlockSpec((1, DH), whole),
               pl.BlockSpec((DH, DH), whole),
               pl.BlockSpec((DH, DOUT), whole),
               pl.BlockSpec((DH, DOUT), whole)]
        ),
        out_specs=[pl.BlockSpec((R, DOUT), blk)] * 2,
        out_shape=[outs_bf, outs_f32],
    )(*s2s, *h1s, recipb, wl2, bl2, wr2, wl3, wr3)


def _final_body(s3_ref, rb_ref, r3_ref, b_ref, bl3_ref, g_ref, be_ref,
                out_ref, psum, csum):
    i = pl.program_id(0)

    @pl.when(i == 0)
    def _init():
        psum[...] = jnp.zeros((G, DOUT), jnp.float32)
        csum[...] = jnp.zeros((G, 1), jnp.float32)

    ssum = s3_ref[0].astype(jnp.float32) + s3_ref[1].astype(jnp.float32)
    out3 = ssum * rb_ref[:, 0:1] + r3_ref[...] + bl3_ref[...]
    bb = b_ref[0]
    gids = lax.broadcasted_iota(jnp.int32, (G, R), 0).astype(jnp.float32)
    onehot = jnp.where(gids == bb, 1.0, 0.0)
    psum[...] += jnp.dot(onehot, out3, preferred_element_type=jnp.float32)
    csum[...] += jnp.sum(onehot, axis=1, keepdims=True)

    @pl.when(i == NBLK - 1)
    def _finish():
        pooled = psum[...] / jnp.maximum(csum[...], 1.0)
        mu = jnp.mean(pooled, axis=1, keepdims=True)
        var = jnp.mean((pooled - mu) ** 2, axis=1, keepdims=True)
        out_ref[...] = ((pooled - mu) * lax.rsqrt(var + 1e-5)
                        * g_ref[...] + be_ref[...])


def _tc_final(s3, recipb, r3, batchf, bl3, ln_g, ln_b):
    blk = lambda i: (i, 0)
    whole = lambda i: (0, 0)
    return pl.pallas_call(
        _final_body,
        grid=(NBLK,),
        in_specs=[
            pl.BlockSpec((2, R, DOUT), lambda i: (0, i, 0)),
            pl.BlockSpec((R, DIN), blk),
            pl.BlockSpec((R, DOUT), blk),
            pl.BlockSpec((1, 1, R), lambda i: (i, 0, 0)),
            pl.BlockSpec((1, DOUT), whole),
            pl.BlockSpec((1, DOUT), whole),
            pl.BlockSpec((1, DOUT), whole),
        ],
        out_specs=pl.BlockSpec((G, DOUT), whole),
        out_shape=jax.ShapeDtypeStruct((G, DOUT), jnp.float32),
        scratch_shapes=[
            pltpu.VMEM((G, DOUT), jnp.float32),
            pltpu.VMEM((G, 1), jnp.float32),
        ],
    )(s3, recipb, r3, batchf, bl3, ln_g, ln_b)


def kernel(x, edge_index, batch, Wl1, bl1, Wr1, Wl2, bl2, Wr2,
           Wl3, bl3, Wr3, ln_g, ln_b):
    f32 = jnp.float32
    src = edge_index[0]
    dst = edge_index[1]
    padidx = (N + (jnp.arange(EPAD - E, dtype=jnp.int32) % (NPAD - N)))
    srcf = jnp.concatenate([src, padidx])
    dstf = jnp.concatenate([dst, padidx])
    srcp = srcf.reshape(EPAD // W, W)
    dstp = dstf.reshape(EPAD // W, W)
    srcp64 = srcf.reshape(EPAD // WL1, WL1)
    dstp64 = dstf.reshape(EPAD // WL1, WL1)

    xa = jnp.concatenate(
        [x, jnp.ones((N, 1), f32), jnp.zeros((N, C1 - DIN - 1), f32)], axis=1)
    xa = jnp.concatenate([xa, jnp.zeros((NPAD - N, C1), f32)], axis=0)
    xa = xa.astype(BF16)
    x_pad = jnp.concatenate([x, jnp.zeros((NPAD - N, DIN), f32)], axis=0)

    zer1 = jnp.zeros((NPAD, C1), BF16)
    zer = jnp.zeros((NPAD, DIN), BF16)

    s1 = _agg_l1(xa, srcp64, dstp64, zer1).reshape(2, NPAD, C1)
    h1s_and_recip = _tc_layer1(s1, x_pad, Wl1, bl1.reshape(1, DH), Wr1)
    h1s, recipb = h1s_and_recip[:4], h1s_and_recip[4]

    s2s = _agg_l2(*h1s, srcp, dstp, zer)
    p3, r3 = _tc_layer2(s2s, h1s, recipb, Wl2, bl2.reshape(1, DH), Wr2,
                        Wl3, Wr3)

    s3 = _agg_l3(p3, srcp, dstp, zer).reshape(2, NPAD, DOUT)
    batchf = jnp.concatenate(
        [batch.astype(f32), jnp.full((NPAD - N,), float(G), f32)]
    ).reshape(NBLK, 1, R)
    out = _tc_final(s3, recipb, r3, batchf, bl3.reshape(1, DOUT),
                    ln_g.reshape(1, DOUT), ln_b.reshape(1, DOUT))
    return out


# speedup vs baseline: 1.1156x; 1.1044x over previous
import functools

import jax
import jax.numpy as jnp
from jax import lax
from jax.experimental import pallas as pl
from jax.experimental.pallas import tpu as pltpu
from jax.experimental.pallas import tpu_sc as plsc

N = 10000
NPAD = 10240
E = 160000
EPAD = 163840
DIN = 128
DH = 512
DOUT = 128
G = 64

NC = 2
NS = 16
W = 128
RPT = NPAD // NS
C1 = DIN + 16

_mesh = plsc.VectorSubcoreMesh(core_axis_name="c", subcore_axis_name="s")


def _edge_loop_db(h_hbm, idxs, wbase, rows_a, rows_b, acc, gsa, gsb, nwin,
                  dget):
    pltpu.async_copy(h_hbm.at[idxs.at[wbase]], rows_a, gsa)

    def body(k, carry):
        j0 = 2 * k
        pltpu.make_async_copy(
            h_hbm.at[idxs.at[wbase + j0]], rows_a, gsa).wait()
        db = pltpu.async_copy(h_hbm.at[idxs.at[wbase + j0 + 1]], rows_b, gsb)
        pltpu.sync_copy(rows_a, acc.at[dget(j0)], add=True)
        db.wait()

        @pl.when(j0 + 2 < nwin)
        def _issue_a():
            pltpu.async_copy(h_hbm.at[idxs.at[wbase + j0 + 2]], rows_a, gsa)

        pltpu.sync_copy(rows_b, acc.at[dget(j0 + 1)], add=True)
        return carry

    lax.fori_loop(0, nwin // 2, body, 0)


def _make_edge_split_agg(C, WL, dt):
    EPW = EPAD // (NC * NS)
    NWIN = EPW // WL

    @functools.partial(
        pl.kernel,
        out_type=jax.ShapeDtypeStruct((NC * NPAD, C), dt),
        mesh=_mesh,
        scratch_types=[
            pltpu.VMEM((NWIN, WL), jnp.int32),
            pltpu.VMEM((NWIN, WL), jnp.int32),
            pltpu.VMEM((WL, C), dt),
            pltpu.VMEM((WL, C), dt),
            pltpu.VMEM_SHARED((NPAD, C), dt),
            pltpu.SemaphoreType.DMA,
            pltpu.SemaphoreType.DMA,
        ],
        compiler_params=pltpu.CompilerParams(use_tc_tiling_on_sc=False),
    )
    def agg(h_hbm, src_hbm, dst_hbm, zer_hbm, out_hbm, idxs, idxd,
            rows_a, rows_b, acc, gsa, gsb):
        c = lax.axis_index("c")
        s = lax.axis_index("s")
        w = s * NC + c
        r0 = s * RPT
        pltpu.sync_copy(zer_hbm.at[pl.ds(r0, RPT)], acc.at[pl.ds(r0, RPT)])
        pltpu.sync_copy(src_hbm.at[pl.ds(w * NWIN, NWIN)], idxs)
        pltpu.sync_copy(dst_hbm.at[pl.ds(w * NWIN, NWIN)], idxd)
        plsc.subcore_barrier()
        _edge_loop_db(h_hbm, idxs, 0, rows_a, rows_b, acc, gsa, gsb,
                      NWIN, lambda j: idxd.at[j])
        plsc.subcore_barrier()
        pltpu.sync_copy(acc.at[pl.ds(r0, RPT)],
                        out_hbm.at[pl.ds(c * NPAD + r0, RPT)])

    return agg


WL1 = 64
BF16 = jnp.bfloat16
_agg_l1 = _make_edge_split_agg(C1, WL1, BF16)
_agg_l3 = _make_edge_split_agg(DOUT, W, BF16)


C2 = 256


def _make_chunk_agg():
    EPT = EPAD // NS
    NWIN = EPT // W

    NHALF = NWIN // 2

    @functools.partial(
        pl.kernel,
        out_type=[jax.ShapeDtypeStruct((NPAD, C2), BF16)] * 2,
        mesh=_mesh,
        scratch_types=[
            pltpu.VMEM((NWIN, W), jnp.int32),
            pltpu.VMEM((NHALF, W), jnp.int32),
            pltpu.VMEM((W, C2), BF16),
            pltpu.VMEM((W, C2), BF16),
            pltpu.VMEM_SHARED((NPAD, C2), BF16),
            pltpu.SemaphoreType.DMA,
            pltpu.SemaphoreType.DMA,
        ],
        compiler_params=pltpu.CompilerParams(use_tc_tiling_on_sc=False),
    )
    def agg2(h0, h1, src_hbm, dst_hbm, zer_hbm,
             o0, o1, idxs, idxd, rows_a, rows_b, acc, gsa, gsb):
        c = lax.axis_index("c")
        s = lax.axis_index("s")
        r0 = s * RPT
        hs = (h0, h1)
        os_ = (o0, o1)
        pltpu.sync_copy(src_hbm.at[pl.ds(s * NWIN, NWIN)], idxs)

        for chunk in range(2):
            h_hbm = hs[chunk]
            out_hbm = os_[chunk]

            @pl.when(c == chunk)
            def _process():
                pltpu.sync_copy(zer_hbm.at[pl.ds(r0, RPT)],
                                acc.at[pl.ds(r0, RPT)])
                plsc.subcore_barrier()
                for half in range(2):
                    pltpu.sync_copy(
                        dst_hbm.at[pl.ds(s * NWIN + half * NHALF, NHALF)],
                        idxd)
                    _edge_loop_db(h_hbm, idxs, half * NHALF, rows_a, rows_b,
                                  acc, gsa, gsb, NHALF, lambda j: idxd.at[j])
                plsc.subcore_barrier()
                pltpu.sync_copy(acc.at[pl.ds(r0, RPT)],
                                out_hbm.at[pl.ds(r0, RPT)])

    return agg2


_agg_l2 = _make_chunk_agg()

R = 256
NBLK = NPAD // R


def _l1_body(s1_ref, x_ref, wl_ref, bl_ref, wr_ref,
             h0_ref, h1_ref, rb_ref):
    ssum = (s1_ref[0].astype(jnp.float32)
            + s1_ref[1].astype(jnp.float32))
    cnt = ssum[:, DIN:DIN + 1]
    recip = 1.0 / jnp.maximum(cnt, 1.0)
    aggv = ssum[:, :DIN] * recip
    h = (jnp.dot(aggv, wl_ref[...], preferred_element_type=jnp.float32)
         + bl_ref[...]
         + jnp.dot(x_ref[...], wr_ref[...], preferred_element_type=jnp.float32))
    h = jnp.maximum(h, 0.0)
    hb = h.astype(jnp.bfloat16)
    h0_ref[...] = hb[:, 0:C2]
    h1_ref[...] = hb[:, C2:DH]
    rb_ref[...] = jnp.broadcast_to(recip, (R, DIN))


def _tc_layer1(s1, x_pad, wl1, bl1, wr1):
    blk = lambda i: (i, 0)
    whole = lambda i: (0, 0)
    outs_bf = jax.ShapeDtypeStruct((NPAD, C2), BF16)
    outs_f32 = jax.ShapeDtypeStruct((NPAD, DIN), jnp.float32)
    return pl.pallas_call(
        _l1_body,
        grid=(NBLK,),
        in_specs=[
            pl.BlockSpec((2, R, C1), lambda i: (0, i, 0)),
            pl.BlockSpec((R, DIN), blk),
            pl.BlockSpec((DIN, DH), whole),
            pl.BlockSpec((1, DH), whole),
            pl.BlockSpec((DIN, DH), whole),
        ],
        out_specs=[pl.BlockSpec((R, C2), blk)] * 2
        + [pl.BlockSpec((R, DIN), blk)],
        out_shape=[outs_bf] * 2 + [outs_f32],
    )(s1, x_pad, wl1, bl1, wr1)


def _l2_body(s20, s21, h10, h11, rb_ref,
             wl2_ref, bl2_ref, wr2_ref, wl3_ref, wr3_ref,
             p3_ref, r3_ref):
    recip = rb_ref[:, 0:1]
    aggv = jnp.concatenate(
        [s20[...], s21[...]], axis=1).astype(jnp.float32) * recip
    h1 = jnp.concatenate(
        [h10[...], h11[...]], axis=1).astype(jnp.float32)
    h2 = (jnp.dot(aggv, wl2_ref[...], preferred_element_type=jnp.float32)
          + bl2_ref[...]
          + jnp.dot(h1, wr2_ref[...], preferred_element_type=jnp.float32))
    h2 = jnp.maximum(h2, 0.0)
    p3_ref[...] = jnp.dot(
        h2, wl3_ref[...], preferred_element_type=jnp.float32
    ).astype(jnp.bfloat16)
    r3_ref[...] = jnp.dot(h2, wr3_ref[...], preferred_element_type=jnp.float32)


def _tc_layer2(s2s, h1s, recipb, wl2, bl2, wr2, wl3, wr3):
    blk = lambda i: (i, 0)
    whole = lambda i: (0, 0)
    outs_bf = jax.ShapeDtypeStruct((NPAD, DOUT), BF16)
    outs_f32 = jax.ShapeDtypeStruct((NPAD, DOUT), jnp.float32)
    return pl.pallas_call(
        _l2_body,
        grid=(NBLK,),
        in_specs=(
            [pl.BlockSpec((R, C2), blk)] * 4
            + [pl.BlockSpec((R, DIN), blk)]
            + [pl.BlockSpec((DH, DH), whole),
               pl.BlockSpec((1, DH), whole),
               pl.BlockSpec((DH, DH), whole),
               pl.BlockSpec((DH, DOUT), whole),
               pl.BlockSpec((DH, DOUT), whole)]
        ),
        out_specs=[pl.BlockSpec((R, DOUT), blk)] * 2,
        out_shape=[outs_bf, outs_f32],
    )(*s2s, *h1s, recipb, wl2, bl2, wr2, wl3, wr3)


def _final_body(s3_ref, rb_ref, r3_ref, b_ref, bl3_ref, g_ref, be_ref,
                out_ref, psum, csum):
    i = pl.program_id(0)

    @pl.when(i == 0)
    def _init():
        psum[...] = jnp.zeros((G, DOUT), jnp.float32)
        csum[...] = jnp.zeros((G, 1), jnp.float32)

    ssum = s3_ref[0].astype(jnp.float32) + s3_ref[1].astype(jnp.float32)
    out3 = ssum * rb_ref[:, 0:1] + r3_ref[...] + bl3_ref[...]
    bb = b_ref[0]
    gids = lax.broadcasted_iota(jnp.int32, (G, R), 0).astype(jnp.float32)
    onehot = jnp.where(gids == bb, 1.0, 0.0)
    psum[...] += jnp.dot(onehot, out3, preferred_element_type=jnp.float32)
    csum[...] += jnp.sum(onehot, axis=1, keepdims=True)

    @pl.when(i == NBLK - 1)
    def _finish():
        pooled = psum[...] / jnp.maximum(csum[...], 1.0)
        mu = jnp.mean(pooled, axis=1, keepdims=True)
        var = jnp.mean((pooled - mu) ** 2, axis=1, keepdims=True)
        out_ref[...] = ((pooled - mu) * lax.rsqrt(var + 1e-5)
                        * g_ref[...] + be_ref[...])


def _tc_final(s3, recipb, r3, batchf, bl3, ln_g, ln_b):
    blk = lambda i: (i, 0)
    whole = lambda i: (0, 0)
    return pl.pallas_call(
        _final_body,
        grid=(NBLK,),
        in_specs=[
            pl.BlockSpec((2, R, DOUT), lambda i: (0, i, 0)),
            pl.BlockSpec((R, DIN), blk),
            pl.BlockSpec((R, DOUT), blk),
            pl.BlockSpec((1, 1, R), lambda i: (i, 0, 0)),
            pl.BlockSpec((1, DOUT), whole),
            pl.BlockSpec((1, DOUT), whole),
            pl.BlockSpec((1, DOUT), whole),
        ],
        out_specs=pl.BlockSpec((G, DOUT), whole),
        out_shape=jax.ShapeDtypeStruct((G, DOUT), jnp.float32),
        scratch_shapes=[
            pltpu.VMEM((G, DOUT), jnp.float32),
            pltpu.VMEM((G, 1), jnp.float32),
        ],
    )(s3, recipb, r3, batchf, bl3, ln_g, ln_b)


def kernel(x, edge_index, batch, Wl1, bl1, Wr1, Wl2, bl2, Wr2,
           Wl3, bl3, Wr3, ln_g, ln_b):
    f32 = jnp.float32
    src = edge_index[0]
    dst = edge_index[1]
    padidx = (N + (jnp.arange(EPAD - E, dtype=jnp.int32) % (NPAD - N)))
    srcf = jnp.concatenate([src, padidx])
    dstf = jnp.concatenate([dst, padidx])
    srcp = srcf.reshape(EPAD // W, W)
    dstp = dstf.reshape(EPAD // W, W)
    srcp64 = srcf.reshape(EPAD // WL1, WL1)
    dstp64 = dstf.reshape(EPAD // WL1, WL1)

    xa = jnp.concatenate(
        [x, jnp.ones((N, 1), f32), jnp.zeros((N, C1 - DIN - 1), f32)], axis=1)
    xa = jnp.concatenate([xa, jnp.zeros((NPAD - N, C1), f32)], axis=0)
    xa = xa.astype(BF16)
    x_pad = jnp.concatenate([x, jnp.zeros((NPAD - N, DIN), f32)], axis=0)

    zer1 = jnp.zeros((NPAD, C1), BF16)
    zer2 = jnp.zeros((NPAD, C2), BF16)
    zer = jnp.zeros((NPAD, DIN), BF16)

    s1 = _agg_l1(xa, srcp64, dstp64, zer1).reshape(2, NPAD, C1)
    h1s_and_recip = _tc_layer1(s1, x_pad, Wl1, bl1.reshape(1, DH), Wr1)
    h1s, recipb = h1s_and_recip[:2], h1s_and_recip[2]

    s2s = _agg_l2(*h1s, srcp, dstp, zer2)
    p3, r3 = _tc_layer2(s2s, h1s, recipb, Wl2, bl2.reshape(1, DH), Wr2,
                        Wl3, Wr3)

    s3 = _agg_l3(p3, srcp, dstp, zer).reshape(2, NPAD, DOUT)
    batchf = jnp.concatenate(
        [batch.astype(f32), jnp.full((NPAD - N,), float(G), f32)]
    ).reshape(NBLK, 1, R)
    out = _tc_final(s3, recipb, r3, batchf, bl3.reshape(1, DOUT),
                    ln_g.reshape(1, DOUT), ln_b.reshape(1, DOUT))
    return out


# speedup vs baseline: 1.1722x; 1.0507x over previous
import functools

import jax
import jax.numpy as jnp
from jax import lax
from jax.experimental import pallas as pl
from jax.experimental.pallas import tpu as pltpu
from jax.experimental.pallas import tpu_sc as plsc

N = 10000
NPAD = 10240
E = 160000
EPAD = 163840
DIN = 128
DH = 512
DOUT = 128
G = 64

NC = 2
NS = 16
W = 128
RPT = NPAD // NS
C1 = DIN + 16

_mesh = plsc.VectorSubcoreMesh(core_axis_name="c", subcore_axis_name="s")


def _edge_loop_db(h_hbm, idxs, wbase, rows_a, rows_b, acc, gsa, gsb, nwin,
                  dget):
    pltpu.async_copy(h_hbm.at[idxs.at[wbase]], rows_a, gsa)

    def body(k, carry):
        j0 = 2 * k
        pltpu.make_async_copy(
            h_hbm.at[idxs.at[wbase + j0]], rows_a, gsa).wait()
        db = pltpu.async_copy(h_hbm.at[idxs.at[wbase + j0 + 1]], rows_b, gsb)
        pltpu.sync_copy(rows_a, acc.at[dget(j0)], add=True)
        db.wait()

        @pl.when(j0 + 2 < nwin)
        def _issue_a():
            pltpu.async_copy(h_hbm.at[idxs.at[wbase + j0 + 2]], rows_a, gsa)

        pltpu.sync_copy(rows_b, acc.at[dget(j0 + 1)], add=True)
        return carry

    lax.fori_loop(0, nwin // 2, body, 0)


def _make_edge_split_agg(C, WL, dt):
    EPW = EPAD // (NC * NS)
    NWIN = EPW // WL

    @functools.partial(
        pl.kernel,
        out_type=jax.ShapeDtypeStruct((NC * NPAD, C), dt),
        mesh=_mesh,
        scratch_types=[
            pltpu.VMEM((NWIN, WL), jnp.int32),
            pltpu.VMEM((NWIN, WL), jnp.int32),
            pltpu.VMEM((WL, C), dt),
            pltpu.VMEM((WL, C), dt),
            pltpu.VMEM_SHARED((NPAD, C), dt),
            pltpu.SemaphoreType.DMA,
            pltpu.SemaphoreType.DMA,
        ],
        compiler_params=pltpu.CompilerParams(use_tc_tiling_on_sc=False),
    )
    def agg(h_hbm, src_hbm, dst_hbm, zer_hbm, out_hbm, idxs, idxd,
            rows_a, rows_b, acc, gsa, gsb):
        c = lax.axis_index("c")
        s = lax.axis_index("s")
        w = s * NC + c
        r0 = s * RPT
        pltpu.sync_copy(zer_hbm.at[pl.ds(r0, RPT)], acc.at[pl.ds(r0, RPT)])
        pltpu.sync_copy(src_hbm.at[pl.ds(w * NWIN, NWIN)], idxs)
        pltpu.sync_copy(dst_hbm.at[pl.ds(w * NWIN, NWIN)], idxd)
        plsc.subcore_barrier()
        _edge_loop_db(h_hbm, idxs, 0, rows_a, rows_b, acc, gsa, gsb,
                      NWIN, lambda j: idxd.at[j])
        plsc.subcore_barrier()
        pltpu.sync_copy(acc.at[pl.ds(r0, RPT)],
                        out_hbm.at[pl.ds(c * NPAD + r0, RPT)])

    return agg


WL1 = 128
BF16 = jnp.bfloat16
_agg_l1 = _make_edge_split_agg(C1, WL1, BF16)
_agg_l3 = _make_edge_split_agg(DOUT, W, BF16)


C2 = 256


def _make_chunk_agg():
    EPT = EPAD // NS
    NWIN = EPT // W

    NHALF = NWIN // 2

    @functools.partial(
        pl.kernel,
        out_type=[jax.ShapeDtypeStruct((NPAD, C2), BF16)] * 2,
        mesh=_mesh,
        scratch_types=[
            pltpu.VMEM((NWIN, W), jnp.int32),
            pltpu.VMEM((NHALF, W), jnp.int32),
            pltpu.VMEM((W, C2), BF16),
            pltpu.VMEM((W, C2), BF16),
            pltpu.VMEM_SHARED((NPAD, C2), BF16),
            pltpu.SemaphoreType.DMA,
            pltpu.SemaphoreType.DMA,
        ],
        compiler_params=pltpu.CompilerParams(use_tc_tiling_on_sc=False),
    )
    def agg2(h0, h1, src_hbm, dst_hbm, zer_hbm,
             o0, o1, idxs, idxd, rows_a, rows_b, acc, gsa, gsb):
        c = lax.axis_index("c")
        s = lax.axis_index("s")
        r0 = s * RPT
        hs = (h0, h1)
        os_ = (o0, o1)
        pltpu.sync_copy(src_hbm.at[pl.ds(s * NWIN, NWIN)], idxs)

        for chunk in range(2):
            h_hbm = hs[chunk]
            out_hbm = os_[chunk]

            @pl.when(c == chunk)
            def _process():
                pltpu.sync_copy(zer_hbm.at[pl.ds(r0, RPT)],
                                acc.at[pl.ds(r0, RPT)])
                plsc.subcore_barrier()
                for half in range(2):
                    pltpu.sync_copy(
                        dst_hbm.at[pl.ds(s * NWIN + half * NHALF, NHALF)],
                        idxd)
                    _edge_loop_db(h_hbm, idxs, half * NHALF, rows_a, rows_b,
                                  acc, gsa, gsb, NHALF, lambda j: idxd.at[j])
                plsc.subcore_barrier()
                pltpu.sync_copy(acc.at[pl.ds(r0, RPT)],
                                out_hbm.at[pl.ds(r0, RPT)])

    return agg2


_agg_l2 = _make_chunk_agg()

R = 256
NBLK = NPAD // R


def _l1_body(s1_ref, x_ref, wl_ref, bl_ref, wr_ref,
             h0_ref, h1_ref, rb_ref):
    ssum = s1_ref[0] + s1_ref[1]
    cnt = ssum[:, DIN:DIN + 1].astype(jnp.float32)
    recip = 1.0 / jnp.maximum(cnt, 1.0)
    h = (jnp.dot(ssum[:, :DIN], wl_ref[...],
                 preferred_element_type=jnp.float32) * recip
         + bl_ref[...]
         + jnp.dot(x_ref[...], wr_ref[...], preferred_element_type=jnp.float32))
    h = jnp.maximum(h, 0.0)
    hb = h.astype(jnp.bfloat16)
    h0_ref[...] = hb[:, 0:C2]
    h1_ref[...] = hb[:, C2:DH]
    rb_ref[...] = jnp.broadcast_to(recip, (R, DIN))


def _tc_layer1(s1, x_pad, wl1, bl1, wr1):
    blk = lambda i: (i, 0)
    whole = lambda i: (0, 0)
    outs_bf = jax.ShapeDtypeStruct((NPAD, C2), BF16)
    outs_f32 = jax.ShapeDtypeStruct((NPAD, DIN), jnp.float32)
    return pl.pallas_call(
        _l1_body,
        grid=(NBLK,),
        in_specs=[
            pl.BlockSpec((2, R, C1), lambda i: (0, i, 0)),
            pl.BlockSpec((R, DIN), blk),
            pl.BlockSpec((DIN, DH), whole),
            pl.BlockSpec((1, DH), whole),
            pl.BlockSpec((DIN, DH), whole),
        ],
        out_specs=[pl.BlockSpec((R, C2), blk)] * 2
        + [pl.BlockSpec((R, DIN), blk)],
        out_shape=[outs_bf] * 2 + [outs_f32],
    )(s1, x_pad, wl1, bl1, wr1)


def _l2_body(s20, s21, h10, h11, rb_ref,
             wl2_ref, bl2_ref, wr2_ref, wl3_ref, wr3_ref,
             p3_ref, r3_ref):
    recip = rb_ref[:, 0:1]
    aggv = jnp.concatenate([s20[...], s21[...]], axis=1)
    h1 = jnp.concatenate([h10[...], h11[...]], axis=1)
    h2 = (jnp.dot(aggv, wl2_ref[...],
                  preferred_element_type=jnp.float32) * recip
          + bl2_ref[...]
          + jnp.dot(h1, wr2_ref[...], preferred_element_type=jnp.float32))
    h2 = jnp.maximum(h2, 0.0).astype(jnp.bfloat16)
    p3_ref[...] = jnp.dot(
        h2, wl3_ref[...], preferred_element_type=jnp.float32
    ).astype(jnp.bfloat16)
    r3_ref[...] = jnp.dot(h2, wr3_ref[...], preferred_element_type=jnp.float32)


def _tc_layer2(s2s, h1s, recipb, wl2, bl2, wr2, wl3, wr3):
    blk = lambda i: (i, 0)
    whole = lambda i: (0, 0)
    outs_bf = jax.ShapeDtypeStruct((NPAD, DOUT), BF16)
    outs_f32 = jax.ShapeDtypeStruct((NPAD, DOUT), jnp.float32)
    return pl.pallas_call(
        _l2_body,
        grid=(NBLK,),
        in_specs=(
            [pl.BlockSpec((R, C2), blk)] * 4
            + [pl.BlockSpec((R, DIN), blk)]
            + [pl.BlockSpec((DH, DH), whole),
               pl.BlockSpec((1, DH), whole),
               pl.BlockSpec((DH, DH), whole),
               pl.BlockSpec((DH, DOUT), whole),
               pl.BlockSpec((DH, DOUT), whole)]
        ),
        out_specs=[pl.BlockSpec((R, DOUT), blk)] * 2,
        out_shape=[outs_bf, outs_f32],
    )(*s2s, *h1s, recipb, wl2, bl2, wr2, wl3, wr3)


def _final_body(s3_ref, rb_ref, r3_ref, b_ref, bl3_ref, g_ref, be_ref,
                out_ref, psum, csum):
    i = pl.program_id(0)

    @pl.when(i == 0)
    def _init():
        psum[...] = jnp.zeros((G, DOUT), jnp.float32)
        csum[...] = jnp.zeros((G, 1), jnp.float32)

    ssum = s3_ref[0].astype(jnp.float32) + s3_ref[1].astype(jnp.float32)
    out3 = ssum * rb_ref[:, 0:1] + r3_ref[...] + bl3_ref[...]
    bb = b_ref[0]
    gids = lax.broadcasted_iota(jnp.int32, (G, R), 0).astype(jnp.float32)
    onehot = jnp.where(gids == bb, 1.0, 0.0)
    psum[...] += jnp.dot(onehot, out3, preferred_element_type=jnp.float32)
    csum[...] += jnp.sum(onehot, axis=1, keepdims=True)

    @pl.when(i == NBLK - 1)
    def _finish():
        pooled = psum[...] / jnp.maximum(csum[...], 1.0)
        mu = jnp.mean(pooled, axis=1, keepdims=True)
        var = jnp.mean((pooled - mu) ** 2, axis=1, keepdims=True)
        out_ref[...] = ((pooled - mu) * lax.rsqrt(var + 1e-5)
                        * g_ref[...] + be_ref[...])


def _tc_final(s3, recipb, r3, batchf, bl3, ln_g, ln_b):
    blk = lambda i: (i, 0)
    whole = lambda i: (0, 0)
    return pl.pallas_call(
        _final_body,
        grid=(NBLK,),
        in_specs=[
            pl.BlockSpec((2, R, DOUT), lambda i: (0, i, 0)),
            pl.BlockSpec((R, DIN), blk),
            pl.BlockSpec((R, DOUT), blk),
            pl.BlockSpec((1, 1, R), lambda i: (i, 0, 0)),
            pl.BlockSpec((1, DOUT), whole),
            pl.BlockSpec((1, DOUT), whole),
            pl.BlockSpec((1, DOUT), whole),
        ],
        out_specs=pl.BlockSpec((G, DOUT), whole),
        out_shape=jax.ShapeDtypeStruct((G, DOUT), jnp.float32),
        scratch_shapes=[
            pltpu.VMEM((G, DOUT), jnp.float32),
            pltpu.VMEM((G, 1), jnp.float32),
        ],
    )(s3, recipb, r3, batchf, bl3, ln_g, ln_b)


def kernel(x, edge_index, batch, Wl1, bl1, Wr1, Wl2, bl2, Wr2,
           Wl3, bl3, Wr3, ln_g, ln_b):
    f32 = jnp.float32
    src = edge_index[0]
    dst = edge_index[1]
    padidx = (N + (jnp.arange(EPAD - E, dtype=jnp.int32) % (NPAD - N)))
    srcf = jnp.concatenate([src, padidx])
    dstf = jnp.concatenate([dst, padidx])
    srcp = srcf.reshape(EPAD // W, W)
    dstp = dstf.reshape(EPAD // W, W)
    srcp64 = srcf.reshape(EPAD // WL1, WL1)
    dstp64 = dstf.reshape(EPAD // WL1, WL1)

    xa = jnp.concatenate(
        [x, jnp.ones((N, 1), f32), jnp.zeros((N, C1 - DIN - 1), f32)], axis=1)
    xa = jnp.concatenate([xa, jnp.zeros((NPAD - N, C1), f32)], axis=0)
    xa = xa.astype(BF16)
    x_pad = jnp.concatenate([x, jnp.zeros((NPAD - N, DIN), f32)],
                            axis=0).astype(BF16)

    zer1 = jnp.zeros((NPAD, C1), BF16)
    zer2 = jnp.zeros((NPAD, C2), BF16)
    zer = jnp.zeros((NPAD, DIN), BF16)

    s1 = _agg_l1(xa, srcp64, dstp64, zer1).reshape(2, NPAD, C1)
    h1s_and_recip = _tc_layer1(s1, x_pad, Wl1.astype(BF16),
                               bl1.reshape(1, DH), Wr1.astype(BF16))
    h1s, recipb = h1s_and_recip[:2], h1s_and_recip[2]

    s2s = _agg_l2(*h1s, srcp, dstp, zer2)
    p3, r3 = _tc_layer2(s2s, h1s, recipb, Wl2.astype(BF16),
                        bl2.reshape(1, DH), Wr2.astype(BF16),
                        Wl3.astype(BF16), Wr3.astype(BF16))

    s3 = _agg_l3(p3, srcp, dstp, zer).reshape(2, NPAD, DOUT)
    batchf = jnp.concatenate(
        [batch.astype(f32), jnp.full((NPAD - N,), float(G), f32)]
    ).reshape(NBLK, 1, R)
    out = _tc_final(s3, recipb, r3, batchf, bl3.reshape(1, DOUT),
                    ln_g.reshape(1, DOUT), ln_b.reshape(1, DOUT))
    return out


# speedup vs baseline: 1.1738x; 1.0014x over previous
import functools

import jax
import jax.numpy as jnp
from jax import lax
from jax.experimental import pallas as pl
from jax.experimental.pallas import tpu as pltpu
from jax.experimental.pallas import tpu_sc as plsc

N = 10000
NPAD = 10240
E = 160000
EPAD = 163840
DIN = 128
DH = 512
DOUT = 128
G = 64

NC = 2
NS = 16
W = 128
RPT = NPAD // NS
C1 = DIN + 16

_mesh = plsc.VectorSubcoreMesh(core_axis_name="c", subcore_axis_name="s")


def _edge_loop_db(h_hbm, idxs, wbase, rows_a, rows_b, acc, gsa, gsb, nwin,
                  dget):
    pltpu.async_copy(h_hbm.at[idxs.at[wbase]], rows_a, gsa)

    def body(k, carry):
        j0 = 2 * k
        pltpu.make_async_copy(
            h_hbm.at[idxs.at[wbase + j0]], rows_a, gsa).wait()
        db = pltpu.async_copy(h_hbm.at[idxs.at[wbase + j0 + 1]], rows_b, gsb)
        pltpu.sync_copy(rows_a, acc.at[dget(j0)], add=True)
        db.wait()

        @pl.when(j0 + 2 < nwin)
        def _issue_a():
            pltpu.async_copy(h_hbm.at[idxs.at[wbase + j0 + 2]], rows_a, gsa)

        pltpu.sync_copy(rows_b, acc.at[dget(j0 + 1)], add=True)
        return carry

    lax.fori_loop(0, nwin // 2, body, 0)


def _make_edge_split_agg(C, WL, dt):
    EPW = EPAD // (NC * NS)
    NWIN = EPW // WL

    @functools.partial(
        pl.kernel,
        out_type=jax.ShapeDtypeStruct((NC, NPAD, C), dt),
        mesh=_mesh,
        scratch_types=[
            pltpu.VMEM((NWIN, WL), jnp.int32),
            pltpu.VMEM((NWIN, WL), jnp.int32),
            pltpu.VMEM((WL, C), dt),
            pltpu.VMEM((WL, C), dt),
            pltpu.VMEM_SHARED((NPAD, C), dt),
            pltpu.SemaphoreType.DMA,
            pltpu.SemaphoreType.DMA,
        ],
        compiler_params=pltpu.CompilerParams(use_tc_tiling_on_sc=False),
    )
    def agg(h_hbm, src_hbm, dst_hbm, zer_hbm, out_hbm, idxs, idxd,
            rows_a, rows_b, acc, gsa, gsb):
        c = lax.axis_index("c")
        s = lax.axis_index("s")
        w = s * NC + c
        r0 = s * RPT
        pltpu.sync_copy(zer_hbm.at[pl.ds(r0, RPT)], acc.at[pl.ds(r0, RPT)])
        pltpu.sync_copy(src_hbm.at[pl.ds(w * NWIN, NWIN)], idxs)
        pltpu.sync_copy(dst_hbm.at[pl.ds(w * NWIN, NWIN)], idxd)
        plsc.subcore_barrier()
        _edge_loop_db(h_hbm, idxs, 0, rows_a, rows_b, acc, gsa, gsb,
                      NWIN, lambda j: idxd.at[j])
        plsc.subcore_barrier()
        pltpu.sync_copy(acc.at[pl.ds(r0, RPT)],
                        out_hbm.at[c, pl.ds(r0, RPT)])

    return agg


WL1 = 128
BF16 = jnp.bfloat16
_agg_l1 = _make_edge_split_agg(C1, WL1, BF16)
_agg_l3 = _make_edge_split_agg(DOUT, W, BF16)


C2 = 256


def _make_chunk_agg():
    EPT = EPAD // NS
    NWIN = EPT // W

    NHALF = NWIN // 2

    @functools.partial(
        pl.kernel,
        out_type=[jax.ShapeDtypeStruct((NPAD, C2), BF16)] * 2,
        mesh=_mesh,
        scratch_types=[
            pltpu.VMEM((NWIN, W), jnp.int32),
            pltpu.VMEM((NHALF, W), jnp.int32),
            pltpu.VMEM((W, C2), BF16),
            pltpu.VMEM((W, C2), BF16),
            pltpu.VMEM_SHARED((NPAD, C2), BF16),
            pltpu.SemaphoreType.DMA,
            pltpu.SemaphoreType.DMA,
        ],
        compiler_params=pltpu.CompilerParams(use_tc_tiling_on_sc=False),
    )
    def agg2(h0, h1, src_hbm, dst_hbm, zer_hbm,
             o0, o1, idxs, idxd, rows_a, rows_b, acc, gsa, gsb):
        c = lax.axis_index("c")
        s = lax.axis_index("s")
        r0 = s * RPT
        hs = (h0, h1)
        os_ = (o0, o1)
        pltpu.sync_copy(src_hbm.at[pl.ds(s * NWIN, NWIN)], idxs)

        for chunk in range(2):
            h_hbm = hs[chunk]
            out_hbm = os_[chunk]

            @pl.when(c == chunk)
            def _process():
                pltpu.sync_copy(zer_hbm.at[pl.ds(r0, RPT)],
                                acc.at[pl.ds(r0, RPT)])
                plsc.subcore_barrier()
                for half in range(2):
                    pltpu.sync_copy(
                        dst_hbm.at[pl.ds(s * NWIN + half * NHALF, NHALF)],
                        idxd)
                    _edge_loop_db(h_hbm, idxs, half * NHALF, rows_a, rows_b,
                                  acc, gsa, gsb, NHALF, lambda j: idxd.at[j])
                plsc.subcore_barrier()
                pltpu.sync_copy(acc.at[pl.ds(r0, RPT)],
                                out_hbm.at[pl.ds(r0, RPT)])

    return agg2


_agg_l2 = _make_chunk_agg()

R = 256
NBLK = NPAD // R


def _l1_body(s1_ref, x_ref, wl_ref, bl_ref, wr_ref,
             h0_ref, h1_ref, rb_ref):
    ssum = s1_ref[0] + s1_ref[1]
    cnt = ssum[:, DIN:DIN + 1].astype(jnp.float32)
    recip = 1.0 / jnp.maximum(cnt, 1.0)
    h = (jnp.dot(ssum[:, :DIN], wl_ref[...].astype(jnp.bfloat16),
                 preferred_element_type=jnp.float32) * recip
         + bl_ref[...]
         + jnp.dot(x_ref[...], wr_ref[...].astype(jnp.bfloat16),
                   preferred_element_type=jnp.float32))
    h = jnp.maximum(h, 0.0)
    hb = h.astype(jnp.bfloat16)
    h0_ref[...] = hb[:, 0:C2]
    h1_ref[...] = hb[:, C2:DH]
    rb_ref[...] = jnp.broadcast_to(recip, (R, DIN)).astype(jnp.bfloat16)


def _tc_layer1(s1, x_pad, wl1, bl1, wr1):
    blk = lambda i: (i, 0)
    whole = lambda i: (0, 0)
    outs_bf = jax.ShapeDtypeStruct((NPAD, C2), BF16)
    outs_rb = jax.ShapeDtypeStruct((NPAD, DIN), BF16)
    return pl.pallas_call(
        _l1_body,
        grid=(NBLK,),
        in_specs=[
            pl.BlockSpec((2, R, C1), lambda i: (0, i, 0)),
            pl.BlockSpec((R, DIN), blk),
            pl.BlockSpec((DIN, DH), whole),
            pl.BlockSpec((1, DH), whole),
            pl.BlockSpec((DIN, DH), whole),
        ],
        out_specs=[pl.BlockSpec((R, C2), blk)] * 2
        + [pl.BlockSpec((R, DIN), blk)],
        out_shape=[outs_bf] * 2 + [outs_rb],
    )(s1, x_pad, wl1, bl1, wr1)


def _l2_body(s20, s21, h10, h11, rb_ref,
             wl2_ref, bl2_ref, wr2_ref, wl3_ref, wr3_ref,
             p3_ref, r3_ref):
    recip = rb_ref[:, 0:1].astype(jnp.float32)
    aggv = jnp.concatenate([s20[...], s21[...]], axis=1)
    h1 = jnp.concatenate([h10[...], h11[...]], axis=1)
    h2 = (jnp.dot(aggv, wl2_ref[...].astype(jnp.bfloat16),
                  preferred_element_type=jnp.float32) * recip
          + bl2_ref[...]
          + jnp.dot(h1, wr2_ref[...].astype(jnp.bfloat16),
                    preferred_element_type=jnp.float32))
    h2 = jnp.maximum(h2, 0.0).astype(jnp.bfloat16)
    p3_ref[...] = jnp.dot(
        h2, wl3_ref[...].astype(jnp.bfloat16),
        preferred_element_type=jnp.float32
    ).astype(jnp.bfloat16)
    r3_ref[...] = jnp.dot(h2, wr3_ref[...].astype(jnp.bfloat16),
                          preferred_element_type=jnp.float32)


def _tc_layer2(s2s, h1s, recipb, wl2, bl2, wr2, wl3, wr3):
    blk = lambda i: (i, 0)
    whole = lambda i: (0, 0)
    outs_bf = jax.ShapeDtypeStruct((NPAD, DOUT), BF16)
    outs_f32 = jax.ShapeDtypeStruct((NPAD, DOUT), jnp.float32)
    return pl.pallas_call(
        _l2_body,
        grid=(NBLK,),
        in_specs=(
            [pl.BlockSpec((R, C2), blk)] * 4
            + [pl.BlockSpec((R, DIN), blk)]
            + [pl.BlockSpec((DH, DH), whole),
               pl.BlockSpec((1, DH), whole),
               pl.BlockSpec((DH, DH), whole),
               pl.BlockSpec((DH, DOUT), whole),
               pl.BlockSpec((DH, DOUT), whole)]
        ),
        out_specs=[pl.BlockSpec((R, DOUT), blk)] * 2,
        out_shape=[outs_bf, outs_f32],
    )(*s2s, *h1s, recipb, wl2, bl2, wr2, wl3, wr3)


def _final_body(s3_ref, rb_ref, r3_ref, b_ref, bl3_ref, g_ref, be_ref,
                out_ref, psum, csum):
    i = pl.program_id(0)

    @pl.when(i == 0)
    def _init():
        psum[...] = jnp.zeros((G, DOUT), jnp.float32)
        csum[...] = jnp.zeros((G, 1), jnp.float32)

    ssum = s3_ref[0].astype(jnp.float32) + s3_ref[1].astype(jnp.float32)
    out3 = (ssum * rb_ref[:, 0:1].astype(jnp.float32)
            + r3_ref[...] + bl3_ref[...])
    bb = b_ref[0]
    gids = lax.broadcasted_iota(jnp.int32, (G, R), 0).astype(jnp.float32)
    onehot = jnp.where(gids == bb, 1.0, 0.0)
    psum[...] += jnp.dot(onehot, out3, preferred_element_type=jnp.float32)
    csum[...] += jnp.sum(onehot, axis=1, keepdims=True)

    @pl.when(i == NBLK - 1)
    def _finish():
        pooled = psum[...] / jnp.maximum(csum[...], 1.0)
        mu = jnp.mean(pooled, axis=1, keepdims=True)
        var = jnp.mean((pooled - mu) ** 2, axis=1, keepdims=True)
        out_ref[...] = ((pooled - mu) * lax.rsqrt(var + 1e-5)
                        * g_ref[...] + be_ref[...])


def _tc_final(s3, recipb, r3, batchf, bl3, ln_g, ln_b):
    blk = lambda i: (i, 0)
    whole = lambda i: (0, 0)
    return pl.pallas_call(
        _final_body,
        grid=(NBLK,),
        in_specs=[
            pl.BlockSpec((2, R, DOUT), lambda i: (0, i, 0)),
            pl.BlockSpec((R, DIN), blk),
            pl.BlockSpec((R, DOUT), blk),
            pl.BlockSpec((1, 1, R), lambda i: (i, 0, 0)),
            pl.BlockSpec((1, DOUT), whole),
            pl.BlockSpec((1, DOUT), whole),
            pl.BlockSpec((1, DOUT), whole),
        ],
        out_specs=pl.BlockSpec((G, DOUT), whole),
        out_shape=jax.ShapeDtypeStruct((G, DOUT), jnp.float32),
        scratch_shapes=[
            pltpu.VMEM((G, DOUT), jnp.float32),
            pltpu.VMEM((G, 1), jnp.float32),
        ],
    )(s3, recipb, r3, batchf, bl3, ln_g, ln_b)


def kernel(x, edge_index, batch, Wl1, bl1, Wr1, Wl2, bl2, Wr2,
           Wl3, bl3, Wr3, ln_g, ln_b):
    f32 = jnp.float32
    src = edge_index[0]
    dst = edge_index[1]
    padidx = (N + (jnp.arange(EPAD - E, dtype=jnp.int32) % (NPAD - N)))
    srcf = jnp.concatenate([src, padidx])
    dstf = jnp.concatenate([dst, padidx])
    srcp = srcf.reshape(EPAD // W, W)
    dstp = dstf.reshape(EPAD // W, W)

    xa = jnp.concatenate(
        [x, jnp.ones((N, 1), f32), jnp.zeros((N, C1 - DIN - 1), f32)], axis=1)
    xa = jnp.concatenate([xa, jnp.zeros((NPAD - N, C1), f32)], axis=0)
    xa = xa.astype(BF16)
    x_pad = jnp.concatenate([x, jnp.zeros((NPAD - N, DIN), f32)],
                            axis=0).astype(BF16)

    zer1 = jnp.zeros((NPAD, C1), BF16)
    zer2 = jnp.zeros((NPAD, C2), BF16)
    zer = jnp.zeros((NPAD, DIN), BF16)

    s1 = _agg_l1(xa, srcp, dstp, zer1)
    h1s_and_recip = _tc_layer1(s1, x_pad, Wl1, bl1.reshape(1, DH), Wr1)
    h1s, recipb = h1s_and_recip[:2], h1s_and_recip[2]

    s2s = _agg_l2(*h1s, srcp, dstp, zer2)
    p3, r3 = _tc_layer2(s2s, h1s, recipb, Wl2, bl2.reshape(1, DH), Wr2,
                        Wl3, Wr3)

    s3 = _agg_l3(p3, srcp, dstp, zer)
    batchf = jnp.concatenate(
        [batch.astype(f32), jnp.full((NPAD - N,), float(G), f32)]
    ).reshape(NBLK, 1, R)
    out = _tc_final(s3, recipb, r3, batchf, bl3.reshape(1, DOUT),
                    ln_g.reshape(1, DOUT), ln_b.reshape(1, DOUT))
    return out


# speedup vs baseline: 1.1811x; 1.0062x over previous
import functools

import jax
import jax.numpy as jnp
from jax import lax
from jax.experimental import pallas as pl
from jax.experimental.pallas import tpu as pltpu
from jax.experimental.pallas import tpu_sc as plsc

N = 10000
NPAD = 10240
E = 160000
EPAD = 163840
DIN = 128
DH = 512
DOUT = 128
G = 64

NC = 2
NS = 16
W = 128
RPT = NPAD // NS
C1 = DIN + 16

_mesh = plsc.VectorSubcoreMesh(core_axis_name="c", subcore_axis_name="s")


def _edge_loop_db(h_hbm, idxs, wbase, rows_a, rows_b, acc, gsa, gsb, nwin,
                  dget):
    pltpu.async_copy(h_hbm.at[idxs.at[wbase]], rows_a, gsa)

    def body(k, carry):
        j0 = 2 * k
        pltpu.make_async_copy(
            h_hbm.at[idxs.at[wbase + j0]], rows_a, gsa).wait()
        db = pltpu.async_copy(h_hbm.at[idxs.at[wbase + j0 + 1]], rows_b, gsb)
        pltpu.sync_copy(rows_a, acc.at[dget(j0)], add=True)
        db.wait()

        @pl.when(j0 + 2 < nwin)
        def _issue_a():
            pltpu.async_copy(h_hbm.at[idxs.at[wbase + j0 + 2]], rows_a, gsa)

        pltpu.sync_copy(rows_b, acc.at[dget(j0 + 1)], add=True)
        return carry

    lax.fori_loop(0, nwin // 2, body, 0)


def _make_edge_split_agg(C, WL, dt, tiled=False):
    EPW = EPAD // (NC * NS)
    NWIN = EPW // WL

    @functools.partial(
        pl.kernel,
        out_type=jax.ShapeDtypeStruct((NC, NPAD, C), dt),
        mesh=_mesh,
        scratch_types=[
            pltpu.VMEM((NWIN, WL), jnp.int32),
            pltpu.VMEM((NWIN, WL), jnp.int32),
            pltpu.VMEM((WL, C), dt),
            pltpu.VMEM((WL, C), dt),
            pltpu.VMEM_SHARED((NPAD, C), dt),
            pltpu.SemaphoreType.DMA,
            pltpu.SemaphoreType.DMA,
        ],
        compiler_params=(None if tiled else
                         pltpu.CompilerParams(use_tc_tiling_on_sc=False)),
    )
    def agg(h_hbm, src_hbm, dst_hbm, zer_hbm, out_hbm, idxs, idxd,
            rows_a, rows_b, acc, gsa, gsb):
        c = lax.axis_index("c")
        s = lax.axis_index("s")
        w = s * NC + c
        r0 = s * RPT
        pltpu.sync_copy(zer_hbm.at[pl.ds(r0, RPT)], acc.at[pl.ds(r0, RPT)])
        pltpu.sync_copy(src_hbm.at[pl.ds(w * NWIN, NWIN)], idxs)
        pltpu.sync_copy(dst_hbm.at[pl.ds(w * NWIN, NWIN)], idxd)
        plsc.subcore_barrier()
        _edge_loop_db(h_hbm, idxs, 0, rows_a, rows_b, acc, gsa, gsb,
                      NWIN, lambda j: idxd.at[j])
        plsc.subcore_barrier()
        pltpu.sync_copy(acc.at[pl.ds(r0, RPT)],
                        out_hbm.at[c, pl.ds(r0, RPT)])

    return agg


WL1 = 128
BF16 = jnp.bfloat16
_agg_l1 = _make_edge_split_agg(C1, WL1, BF16)
_agg_l3 = _make_edge_split_agg(DOUT, W, jnp.float32, tiled=True)


C2 = 256


def _make_chunk_agg():
    EPT = EPAD // NS
    NWIN = EPT // W

    NHALF = NWIN // 2

    @functools.partial(
        pl.kernel,
        out_type=[jax.ShapeDtypeStruct((NPAD, C2), BF16)] * 2,
        mesh=_mesh,
        scratch_types=[
            pltpu.VMEM((NWIN, W), jnp.int32),
            pltpu.VMEM((NHALF, W), jnp.int32),
            pltpu.VMEM((W, C2), BF16),
            pltpu.VMEM((W, C2), BF16),
            pltpu.VMEM_SHARED((NPAD, C2), BF16),
            pltpu.SemaphoreType.DMA,
            pltpu.SemaphoreType.DMA,
        ],
        compiler_params=pltpu.CompilerParams(use_tc_tiling_on_sc=False),
    )
    def agg2(h0, h1, src_hbm, dst_hbm, zer_hbm,
             o0, o1, idxs, idxd, rows_a, rows_b, acc, gsa, gsb):
        c = lax.axis_index("c")
        s = lax.axis_index("s")
        r0 = s * RPT
        hs = (h0, h1)
        os_ = (o0, o1)
        pltpu.sync_copy(src_hbm.at[pl.ds(s * NWIN, NWIN)], idxs)

        for chunk in range(2):
            h_hbm = hs[chunk]
            out_hbm = os_[chunk]

            @pl.when(c == chunk)
            def _process():
                pltpu.sync_copy(zer_hbm.at[pl.ds(r0, RPT)],
                                acc.at[pl.ds(r0, RPT)])
                plsc.subcore_barrier()
                for half in range(2):
                    pltpu.sync_copy(
                        dst_hbm.at[pl.ds(s * NWIN + half * NHALF, NHALF)],
                        idxd)
                    _edge_loop_db(h_hbm, idxs, half * NHALF, rows_a, rows_b,
                                  acc, gsa, gsb, NHALF, lambda j: idxd.at[j])
                plsc.subcore_barrier()
                pltpu.sync_copy(acc.at[pl.ds(r0, RPT)],
                                out_hbm.at[pl.ds(r0, RPT)])

    return agg2


_agg_l2 = _make_chunk_agg()

R = 256
NBLK = NPAD // R


def _l1_body(s1_ref, x_ref, wl_ref, bl_ref, wr_ref,
             h0_ref, h1_ref, rb_ref):
    ssum = s1_ref[0] + s1_ref[1]
    cnt = ssum[:, DIN:DIN + 1].astype(jnp.float32)
    recip = 1.0 / jnp.maximum(cnt, 1.0)
    h = (jnp.dot(ssum[:, :DIN], wl_ref[...].astype(jnp.bfloat16),
                 preferred_element_type=jnp.float32) * recip
         + bl_ref[...]
         + jnp.dot(x_ref[...], wr_ref[...].astype(jnp.bfloat16),
                   preferred_element_type=jnp.float32))
    h = jnp.maximum(h, 0.0)
    hb = h.astype(jnp.bfloat16)
    h0_ref[...] = hb[:, 0:C2]
    h1_ref[...] = hb[:, C2:DH]
    rb_ref[...] = jnp.broadcast_to(recip, (R, DIN)).astype(jnp.bfloat16)


def _tc_layer1(s1, x_pad, wl1, bl1, wr1):
    blk = lambda i: (i, 0)
    whole = lambda i: (0, 0)
    outs_bf = jax.ShapeDtypeStruct((NPAD, C2), BF16)
    outs_rb = jax.ShapeDtypeStruct((NPAD, DIN), BF16)
    return pl.pallas_call(
        _l1_body,
        grid=(NBLK,),
        in_specs=[
            pl.BlockSpec((2, R, C1), lambda i: (0, i, 0)),
            pl.BlockSpec((R, DIN), blk),
            pl.BlockSpec((DIN, DH), whole),
            pl.BlockSpec((1, DH), whole),
            pl.BlockSpec((DIN, DH), whole),
        ],
        out_specs=[pl.BlockSpec((R, C2), blk)] * 2
        + [pl.BlockSpec((R, DIN), blk)],
        out_shape=[outs_bf] * 2 + [outs_rb],
    )(s1, x_pad, wl1, bl1, wr1)


def _l2_body(s20, s21, h10, h11, rb_ref,
             wl2_ref, bl2_ref, wr2_ref, wl3_ref, wr3_ref,
             p3_ref, r3_ref):
    recip = rb_ref[:, 0:1].astype(jnp.float32)
    aggv = jnp.concatenate([s20[...], s21[...]], axis=1)
    h1 = jnp.concatenate([h10[...], h11[...]], axis=1)
    h2 = (jnp.dot(aggv, wl2_ref[...].astype(jnp.bfloat16),
                  preferred_element_type=jnp.float32) * recip
          + bl2_ref[...]
          + jnp.dot(h1, wr2_ref[...].astype(jnp.bfloat16),
                    preferred_element_type=jnp.float32))
    h2 = jnp.maximum(h2, 0.0).astype(jnp.bfloat16)
    p3_ref[...] = jnp.dot(h2, wl3_ref[...].astype(jnp.bfloat16),
                          preferred_element_type=jnp.float32)
    r3_ref[...] = jnp.dot(h2, wr3_ref[...].astype(jnp.bfloat16),
                          preferred_element_type=jnp.float32)


def _tc_layer2(s2s, h1s, recipb, wl2, bl2, wr2, wl3, wr3):
    blk = lambda i: (i, 0)
    whole = lambda i: (0, 0)
    outs_f32 = jax.ShapeDtypeStruct((NPAD, DOUT), jnp.float32)
    return pl.pallas_call(
        _l2_body,
        grid=(NBLK,),
        in_specs=(
            [pl.BlockSpec((R, C2), blk)] * 4
            + [pl.BlockSpec((R, DIN), blk)]
            + [pl.BlockSpec((DH, DH), whole),
               pl.BlockSpec((1, DH), whole),
               pl.BlockSpec((DH, DH), whole),
               pl.BlockSpec((DH, DOUT), whole),
               pl.BlockSpec((DH, DOUT), whole)]
        ),
        out_specs=[pl.BlockSpec((R, DOUT), blk)] * 2,
        out_shape=[outs_f32, outs_f32],
    )(*s2s, *h1s, recipb, wl2, bl2, wr2, wl3, wr3)


def _final_body(s3_ref, rb_ref, r3_ref, b_ref, bl3_ref, g_ref, be_ref,
                out_ref, psum, csum):
    i = pl.program_id(0)

    @pl.when(i == 0)
    def _init():
        psum[...] = jnp.zeros((G, DOUT), jnp.float32)
        csum[...] = jnp.zeros((G, 1), jnp.float32)

    ssum = s3_ref[0].astype(jnp.float32) + s3_ref[1].astype(jnp.float32)
    out3 = (ssum * rb_ref[:, 0:1].astype(jnp.float32)
            + r3_ref[...] + bl3_ref[...])
    bb = b_ref[0]
    gids = lax.broadcasted_iota(jnp.int32, (G, R), 0).astype(jnp.float32)
    onehot = jnp.where(gids == bb, 1.0, 0.0)
    psum[...] += jnp.dot(onehot, out3, preferred_element_type=jnp.float32)
    csum[...] += jnp.sum(onehot, axis=1, keepdims=True)

    @pl.when(i == NBLK - 1)
    def _finish():
        pooled = psum[...] / jnp.maximum(csum[...], 1.0)
        mu = jnp.mean(pooled, axis=1, keepdims=True)
        var = jnp.mean((pooled - mu) ** 2, axis=1, keepdims=True)
        out_ref[...] = ((pooled - mu) * lax.rsqrt(var + 1e-5)
                        * g_ref[...] + be_ref[...])


def _tc_final(s3, recipb, r3, batchf, bl3, ln_g, ln_b):
    blk = lambda i: (i, 0)
    whole = lambda i: (0, 0)
    return pl.pallas_call(
        _final_body,
        grid=(NBLK,),
        in_specs=[
            pl.BlockSpec((2, R, DOUT), lambda i: (0, i, 0)),
            pl.BlockSpec((R, DIN), blk),
            pl.BlockSpec((R, DOUT), blk),
            pl.BlockSpec((1, 1, R), lambda i: (i, 0, 0)),
            pl.BlockSpec((1, DOUT), whole),
            pl.BlockSpec((1, DOUT), whole),
            pl.BlockSpec((1, DOUT), whole),
        ],
        out_specs=pl.BlockSpec((G, DOUT), whole),
        out_shape=jax.ShapeDtypeStruct((G, DOUT), jnp.float32),
        scratch_shapes=[
            pltpu.VMEM((G, DOUT), jnp.float32),
            pltpu.VMEM((G, 1), jnp.float32),
        ],
    )(s3, recipb, r3, batchf, bl3, ln_g, ln_b)


def kernel(x, edge_index, batch, Wl1, bl1, Wr1, Wl2, bl2, Wr2,
           Wl3, bl3, Wr3, ln_g, ln_b):
    f32 = jnp.float32
    src = edge_index[0]
    dst = edge_index[1]
    padidx = (N + (jnp.arange(EPAD - E, dtype=jnp.int32) % (NPAD - N)))
    srcf = jnp.concatenate([src, padidx])
    dstf = jnp.concatenate([dst, padidx])
    srcp = srcf.reshape(EPAD // W, W)
    dstp = dstf.reshape(EPAD // W, W)

    xa = jnp.concatenate(
        [x, jnp.ones((N, 1), f32), jnp.zeros((N, C1 - DIN - 1), f32)], axis=1)
    xa = jnp.concatenate([xa, jnp.zeros((NPAD - N, C1), f32)], axis=0)
    xa = xa.astype(BF16)
    x_pad = jnp.concatenate([x, jnp.zeros((NPAD - N, DIN), f32)],
                            axis=0).astype(BF16)

    zer1 = jnp.zeros((NPAD, C1), BF16)
    zer2 = jnp.zeros((NPAD, C2), BF16)
    zer3 = jnp.zeros((NPAD, DIN), f32)

    s1 = _agg_l1(xa, srcp, dstp, zer1)
    h1s_and_recip = _tc_layer1(s1, x_pad, Wl1, bl1.reshape(1, DH), Wr1)
    h1s, recipb = h1s_and_recip[:2], h1s_and_recip[2]

    s2s = _agg_l2(*h1s, srcp, dstp, zer2)
    p3, r3 = _tc_layer2(s2s, h1s, recipb, Wl2, bl2.reshape(1, DH), Wr2,
                        Wl3, Wr3)

    s3 = _agg_l3(p3, srcp, dstp, zer3)
    batchf = jnp.concatenate(
        [batch.astype(f32), jnp.full((NPAD - N,), float(G), f32)]
    ).reshape(NBLK, 1, R)
    out = _tc_final(s3, recipb, r3, batchf, bl3.reshape(1, DOUT),
                    ln_g.reshape(1, DOUT), ln_b.reshape(1, DOUT))
    return out


# speedup vs baseline: 1.2426x; 1.0520x over previous
import functools

import jax
import jax.numpy as jnp
from jax import lax
from jax.experimental import pallas as pl
from jax.experimental.pallas import tpu as pltpu
from jax.experimental.pallas import tpu_sc as plsc

N = 10000
NPAD = 10240
E = 160000
EPAD = 163840
DIN = 128
DH = 512
DOUT = 128
G = 64

NC = 2
NS = 16
W = 128
RPT = NPAD // NS
C1 = DIN + 16

_mesh = plsc.VectorSubcoreMesh(core_axis_name="c", subcore_axis_name="s")


def _edge_loop_db(h_hbm, idxs, wbase, rows_a, rows_b, acc, gsa, gsb, nwin,
                  dget):
    pltpu.async_copy(h_hbm.at[idxs.at[wbase]], rows_a, gsa)

    def body(k, carry):
        j0 = 2 * k
        pltpu.make_async_copy(
            h_hbm.at[idxs.at[wbase + j0]], rows_a, gsa).wait()
        db = pltpu.async_copy(h_hbm.at[idxs.at[wbase + j0 + 1]], rows_b, gsb)
        pltpu.sync_copy(rows_a, acc.at[dget(j0)], add=True)
        db.wait()

        @pl.when(j0 + 2 < nwin)
        def _issue_a():
            pltpu.async_copy(h_hbm.at[idxs.at[wbase + j0 + 2]], rows_a, gsa)

        pltpu.sync_copy(rows_b, acc.at[dget(j0 + 1)], add=True)
        return carry

    lax.fori_loop(0, nwin // 2, body, 0)


def _edge_loop_q(h_hbm, idxs, wbase, rows, acc, gs, ss, nwin, dget):
    nb = 4

    def body(k, carry):
        for b in range(nb):
            i = nb * k + b

            @pl.when((i >= nb) & (i < nwin))
            def _wait_s():
                pltpu.make_async_copy(
                    rows[b], acc.at[dget(i - nb)], ss[b]).wait()

            @pl.when(i < nwin)
            def _issue_g():
                pltpu.async_copy(h_hbm.at[idxs.at[wbase + i]], rows[b],
                                 gs[b])

            b2 = (b + 2) % nb

            @pl.when((i >= 2) & (i < nwin + 2))
            def _scatter():
                j = i - 2
                pltpu.make_async_copy(
                    h_hbm.at[idxs.at[wbase + j]], rows[b2], gs[b2]).wait()
                pltpu.async_copy(rows[b2], acc.at[dget(j)], ss[b2],
                                 add=True)
        return carry

    lax.fori_loop(0, (nwin + 2 + nb - 1) // nb + 1, body, 0)
    for t in range(nb):
        wj = nwin - nb + t
        pltpu.make_async_copy(rows[wj % nb], acc.at[dget(wj)],
                              ss[wj % nb]).wait()


def _make_edge_split_agg(C, WL, dt, tiled=False):
    EPW = EPAD // (NC * NS)
    NWIN = EPW // WL

    @functools.partial(
        pl.kernel,
        out_type=jax.ShapeDtypeStruct((NC, NPAD, C), dt),
        mesh=_mesh,
        scratch_types=[
            pltpu.VMEM((NWIN, WL), jnp.int32),
            pltpu.VMEM((NWIN, WL), jnp.int32),
            pltpu.VMEM((WL, C), dt),
            pltpu.VMEM((WL, C), dt),
            pltpu.VMEM_SHARED((NPAD, C), dt),
            pltpu.SemaphoreType.DMA,
            pltpu.SemaphoreType.DMA,
        ],
        compiler_params=(None if tiled else
                         pltpu.CompilerParams(use_tc_tiling_on_sc=False)),
    )
    def agg(h_hbm, src_hbm, dst_hbm, zer_hbm, out_hbm, idxs, idxd,
            rows_a, rows_b, acc, gsa, gsb):
        c = lax.axis_index("c")
        s = lax.axis_index("s")
        w = s * NC + c
        r0 = s * RPT
        pltpu.sync_copy(zer_hbm.at[pl.ds(r0, RPT)], acc.at[pl.ds(r0, RPT)])
        pltpu.sync_copy(src_hbm.at[pl.ds(w * NWIN, NWIN)], idxs)
        pltpu.sync_copy(dst_hbm.at[pl.ds(w * NWIN, NWIN)], idxd)
        plsc.subcore_barrier()
        _edge_loop_db(h_hbm, idxs, 0, rows_a, rows_b, acc, gsa, gsb,
                      NWIN, lambda j: idxd.at[j])
        plsc.subcore_barrier()
        pltpu.sync_copy(acc.at[pl.ds(r0, RPT)],
                        out_hbm.at[c, pl.ds(r0, RPT)])

    return agg


WL1 = 128
BF16 = jnp.bfloat16
_agg_l1 = _make_edge_split_agg(C1, WL1, BF16)
_agg_l3 = _make_edge_split_agg(DOUT, W, jnp.float32, tiled=True)


C2 = 256


def _make_chunk_agg():
    W2 = 64
    EPT = EPAD // NS
    NWIN = EPT // W2

    NHALF = NWIN // 2

    @functools.partial(
        pl.kernel,
        out_type=[jax.ShapeDtypeStruct((NPAD, C2), BF16)] * 2,
        mesh=_mesh,
        scratch_types=[
            pltpu.VMEM((NWIN, W2), jnp.int32),
            pltpu.VMEM((NHALF, W2), jnp.int32),
            pltpu.VMEM((W2, C2), BF16),
            pltpu.VMEM((W2, C2), BF16),
            pltpu.VMEM((W2, C2), BF16),
            pltpu.VMEM((W2, C2), BF16),
            pltpu.VMEM_SHARED((NPAD, C2), BF16),
            pltpu.SemaphoreType.DMA,
            pltpu.SemaphoreType.DMA,
            pltpu.SemaphoreType.DMA,
            pltpu.SemaphoreType.DMA,
            pltpu.SemaphoreType.DMA,
            pltpu.SemaphoreType.DMA,
            pltpu.SemaphoreType.DMA,
            pltpu.SemaphoreType.DMA,
        ],
        compiler_params=pltpu.CompilerParams(use_tc_tiling_on_sc=False),
    )
    def agg2(h0, h1, src_hbm, dst_hbm, zer_hbm,
             o0, o1, idxs, idxd, ra, rb, rc, rd,
             acc, g0, g1, g2, g3, s0, s1_, s2_, s3_):
        c = lax.axis_index("c")
        s = lax.axis_index("s")
        r0 = s * RPT
        hs = (h0, h1)
        os_ = (o0, o1)
        rows = (ra, rb, rc, rd)
        gs = (g0, g1, g2, g3)
        ss = (s0, s1_, s2_, s3_)
        pltpu.sync_copy(src_hbm.at[pl.ds(s * NWIN, NWIN)], idxs)

        for chunk in range(2):
            h_hbm = hs[chunk]
            out_hbm = os_[chunk]

            @pl.when(c == chunk)
            def _process():
                pltpu.sync_copy(zer_hbm.at[pl.ds(r0, RPT)],
                                acc.at[pl.ds(r0, RPT)])
                plsc.subcore_barrier()
                for half in range(2):
                    pltpu.sync_copy(
                        dst_hbm.at[pl.ds(s * NWIN + half * NHALF, NHALF)],
                        idxd)
                    _edge_loop_q(h_hbm, idxs, half * NHALF, rows,
                                 acc, gs, ss, NHALF, lambda j: idxd.at[j])
                plsc.subcore_barrier()
                pltpu.sync_copy(acc.at[pl.ds(r0, RPT)],
                                out_hbm.at[pl.ds(r0, RPT)])

    return agg2


_agg_l2 = _make_chunk_agg()

R = 256
NBLK = NPAD // R


def _l1_body(s1_ref, x_ref, wl_ref, bl_ref, wr_ref,
             h0_ref, h1_ref, rb_ref):
    ssum = s1_ref[0] + s1_ref[1]
    cnt = ssum[:, DIN:DIN + 1].astype(jnp.float32)
    recip = 1.0 / jnp.maximum(cnt, 1.0)
    h = (jnp.dot(ssum[:, :DIN], wl_ref[...].astype(jnp.bfloat16),
                 preferred_element_type=jnp.float32) * recip
         + bl_ref[...]
         + jnp.dot(x_ref[...], wr_ref[...].astype(jnp.bfloat16),
                   preferred_element_type=jnp.float32))
    h = jnp.maximum(h, 0.0)
    hb = h.astype(jnp.bfloat16)
    h0_ref[...] = hb[:, 0:C2]
    h1_ref[...] = hb[:, C2:DH]
    rb_ref[...] = jnp.broadcast_to(recip, (R, DIN)).astype(jnp.bfloat16)


def _tc_layer1(s1, x_pad, wl1, bl1, wr1):
    blk = lambda i: (i, 0)
    whole = lambda i: (0, 0)
    outs_bf = jax.ShapeDtypeStruct((NPAD, C2), BF16)
    outs_rb = jax.ShapeDtypeStruct((NPAD, DIN), BF16)
    return pl.pallas_call(
        _l1_body,
        grid=(NBLK,),
        in_specs=[
            pl.BlockSpec((2, R, C1), lambda i: (0, i, 0)),
            pl.BlockSpec((R, DIN), blk),
            pl.BlockSpec((DIN, DH), whole),
            pl.BlockSpec((1, DH), whole),
            pl.BlockSpec((DIN, DH), whole),
        ],
        out_specs=[pl.BlockSpec((R, C2), blk)] * 2
        + [pl.BlockSpec((R, DIN), blk)],
        out_shape=[outs_bf] * 2 + [outs_rb],
    )(s1, x_pad, wl1, bl1, wr1)


def _l2_body(s20, s21, h10, h11, rb_ref,
             wl2_ref, bl2_ref, wr2_ref, wl3_ref, wr3_ref,
             p3_ref, r3_ref):
    recip = rb_ref[:, 0:1].astype(jnp.float32)
    aggv = jnp.concatenate([s20[...], s21[...]], axis=1)
    h1 = jnp.concatenate([h10[...], h11[...]], axis=1)
    h2 = (jnp.dot(aggv, wl2_ref[...].astype(jnp.bfloat16),
                  preferred_element_type=jnp.float32) * recip
          + bl2_ref[...]
          + jnp.dot(h1, wr2_ref[...].astype(jnp.bfloat16),
                    preferred_element_type=jnp.float32))
    h2 = jnp.maximum(h2, 0.0).astype(jnp.bfloat16)
    p3_ref[...] = jnp.dot(h2, wl3_ref[...].astype(jnp.bfloat16),
                          preferred_element_type=jnp.float32)
    r3_ref[...] = jnp.dot(h2, wr3_ref[...].astype(jnp.bfloat16),
                          preferred_element_type=jnp.float32)


def _tc_layer2(s2s, h1s, recipb, wl2, bl2, wr2, wl3, wr3):
    blk = lambda i: (i, 0)
    whole = lambda i: (0, 0)
    outs_f32 = jax.ShapeDtypeStruct((NPAD, DOUT), jnp.float32)
    return pl.pallas_call(
        _l2_body,
        grid=(NBLK,),
        in_specs=(
            [pl.BlockSpec((R, C2), blk)] * 4
            + [pl.BlockSpec((R, DIN), blk)]
            + [pl.BlockSpec((DH, DH), whole),
               pl.BlockSpec((1, DH), whole),
               pl.BlockSpec((DH, DH), whole),
               pl.BlockSpec((DH, DOUT), whole),
               pl.BlockSpec((DH, DOUT), whole)]
        ),
        out_specs=[pl.BlockSpec((R, DOUT), blk)] * 2,
        out_shape=[outs_f32, outs_f32],
    )(*s2s, *h1s, recipb, wl2, bl2, wr2, wl3, wr3)


def _final_body(s3_ref, rb_ref, r3_ref, b_ref, bl3_ref, g_ref, be_ref,
                out_ref, psum, csum):
    i = pl.program_id(0)

    @pl.when(i == 0)
    def _init():
        psum[...] = jnp.zeros((G, DOUT), jnp.float32)
        csum[...] = jnp.zeros((G, 1), jnp.float32)

    ssum = s3_ref[0].astype(jnp.float32) + s3_ref[1].astype(jnp.float32)
    out3 = (ssum * rb_ref[:, 0:1].astype(jnp.float32)
            + r3_ref[...] + bl3_ref[...])
    bb = b_ref[0]
    gids = lax.broadcasted_iota(jnp.int32, (G, R), 0).astype(jnp.float32)
    onehot = jnp.where(gids == bb, 1.0, 0.0)
    psum[...] += jnp.dot(onehot, out3, preferred_element_type=jnp.float32)
    csum[...] += jnp.sum(onehot, axis=1, keepdims=True)

    @pl.when(i == NBLK - 1)
    def _finish():
        pooled = psum[...] / jnp.maximum(csum[...], 1.0)
        mu = jnp.mean(pooled, axis=1, keepdims=True)
        var = jnp.mean((pooled - mu) ** 2, axis=1, keepdims=True)
        out_ref[...] = ((pooled - mu) * lax.rsqrt(var + 1e-5)
                        * g_ref[...] + be_ref[...])


def _tc_final(s3, recipb, r3, batchf, bl3, ln_g, ln_b):
    blk = lambda i: (i, 0)
    whole = lambda i: (0, 0)
    return pl.pallas_call(
        _final_body,
        grid=(NBLK,),
        in_specs=[
            pl.BlockSpec((2, R, DOUT), lambda i: (0, i, 0)),
            pl.BlockSpec((R, DIN), blk),
            pl.BlockSpec((R, DOUT), blk),
            pl.BlockSpec((1, 1, R), lambda i: (i, 0, 0)),
            pl.BlockSpec((1, DOUT), whole),
            pl.BlockSpec((1, DOUT), whole),
            pl.BlockSpec((1, DOUT), whole),
        ],
        out_specs=pl.BlockSpec((G, DOUT), whole),
        out_shape=jax.ShapeDtypeStruct((G, DOUT), jnp.float32),
        scratch_shapes=[
            pltpu.VMEM((G, DOUT), jnp.float32),
            pltpu.VMEM((G, 1), jnp.float32),
        ],
    )(s3, recipb, r3, batchf, bl3, ln_g, ln_b)


def kernel(x, edge_index, batch, Wl1, bl1, Wr1, Wl2, bl2, Wr2,
           Wl3, bl3, Wr3, ln_g, ln_b):
    f32 = jnp.float32
    src = edge_index[0]
    dst = edge_index[1]
    padidx = (N + (jnp.arange(EPAD - E, dtype=jnp.int32) % (NPAD - N)))
    srcf = jnp.concatenate([src, padidx])
    dstf = jnp.concatenate([dst, padidx])
    srcp = srcf.reshape(EPAD // W, W)
    dstp = dstf.reshape(EPAD // W, W)
    srcp64 = srcf.reshape(EPAD // 64, 64)
    dstp64 = dstf.reshape(EPAD // 64, 64)

    xa = jnp.concatenate(
        [x, jnp.ones((N, 1), f32), jnp.zeros((N, C1 - DIN - 1), f32)], axis=1)
    xa = jnp.concatenate([xa, jnp.zeros((NPAD - N, C1), f32)], axis=0)
    xa = xa.astype(BF16)
    x_pad = jnp.concatenate([x, jnp.zeros((NPAD - N, DIN), f32)],
                            axis=0).astype(BF16)

    zer1 = jnp.zeros((NPAD, C1), BF16)
    zer2 = jnp.zeros((NPAD, C2), BF16)
    zer3 = jnp.zeros((NPAD, DIN), f32)

    s1 = _agg_l1(xa, srcp, dstp, zer1)
    h1s_and_recip = _tc_layer1(s1, x_pad, Wl1, bl1.reshape(1, DH), Wr1)
    h1s, recipb = h1s_and_recip[:2], h1s_and_recip[2]

    s2s = _agg_l2(*h1s, srcp64, dstp64, zer2)
    p3, r3 = _tc_layer2(s2s, h1s, recipb, Wl2, bl2.reshape(1, DH), Wr2,
                        Wl3, Wr3)

    s3 = _agg_l3(p3, srcp, dstp, zer3)
    batchf = jnp.concatenate(
        [batch.astype(f32), jnp.full((NPAD - N,), float(G), f32)]
    ).reshape(NBLK, 1, R)
    out = _tc_final(s3, recipb, r3, batchf, bl3.reshape(1, DOUT),
                    ln_g.reshape(1, DOUT), ln_b.reshape(1, DOUT))
    return out


# speedup vs baseline: 1.3060x; 1.0510x over previous
import functools

import jax
import jax.numpy as jnp
from jax import lax
from jax.experimental import pallas as pl
from jax.experimental.pallas import tpu as pltpu
from jax.experimental.pallas import tpu_sc as plsc

N = 10000
NPAD = 10240
E = 160000
EPAD = 163840
DIN = 128
DH = 512
DOUT = 128
G = 64

NC = 2
NS = 16
W = 128
RPT = NPAD // NS
C1 = DIN + 16

_mesh = plsc.VectorSubcoreMesh(core_axis_name="c", subcore_axis_name="s")


def _edge_loop_db(h_hbm, idxs, wbase, rows_a, rows_b, acc, gsa, gsb, nwin,
                  dget):
    pltpu.async_copy(h_hbm.at[idxs.at[wbase]], rows_a, gsa)

    def body(k, carry):
        j0 = 2 * k
        pltpu.make_async_copy(
            h_hbm.at[idxs.at[wbase + j0]], rows_a, gsa).wait()
        db = pltpu.async_copy(h_hbm.at[idxs.at[wbase + j0 + 1]], rows_b, gsb)
        pltpu.sync_copy(rows_a, acc.at[dget(j0)], add=True)
        db.wait()

        @pl.when(j0 + 2 < nwin)
        def _issue_a():
            pltpu.async_copy(h_hbm.at[idxs.at[wbase + j0 + 2]], rows_a, gsa)

        pltpu.sync_copy(rows_b, acc.at[dget(j0 + 1)], add=True)
        return carry

    lax.fori_loop(0, nwin // 2, body, 0)


def _edge_loop_q(h_hbm, idxs, wbase, rows, acc, gs, ss, nwin, dget):
    nb = 4

    def body(k, carry):
        for b in range(nb):
            i = nb * k + b

            @pl.when((i >= nb) & (i < nwin))
            def _wait_s():
                pltpu.make_async_copy(
                    rows[b], acc.at[dget(i - nb)], ss[b]).wait()

            @pl.when(i < nwin)
            def _issue_g():
                pltpu.async_copy(h_hbm.at[idxs.at[wbase + i]], rows[b],
                                 gs[b])

            b2 = (b + 2) % nb

            @pl.when((i >= 2) & (i < nwin + 2))
            def _scatter():
                j = i - 2
                pltpu.make_async_copy(
                    h_hbm.at[idxs.at[wbase + j]], rows[b2], gs[b2]).wait()
                pltpu.async_copy(rows[b2], acc.at[dget(j)], ss[b2],
                                 add=True)
        return carry

    lax.fori_loop(0, (nwin + 2 + nb - 1) // nb + 1, body, 0)
    for t in range(nb):
        wj = nwin - nb + t
        pltpu.make_async_copy(rows[wj % nb], acc.at[dget(wj)],
                              ss[wj % nb]).wait()


def _make_edge_split_agg(C, WL, dt, tiled=False):
    EPW = EPAD // (NC * NS)
    NWIN = EPW // WL
    NHALF = NWIN // 2

    @functools.partial(
        pl.kernel,
        out_type=jax.ShapeDtypeStruct((NC, NPAD, C), dt),
        mesh=_mesh,
        scratch_types=[
            pltpu.VMEM((NWIN, WL), jnp.int32),
            pltpu.VMEM((NHALF, WL), jnp.int32),
            pltpu.VMEM((WL, C), dt),
            pltpu.VMEM((WL, C), dt),
            pltpu.VMEM((WL, C), dt),
            pltpu.VMEM((WL, C), dt),
            pltpu.VMEM_SHARED((NPAD, C), dt),
            pltpu.SemaphoreType.DMA,
            pltpu.SemaphoreType.DMA,
            pltpu.SemaphoreType.DMA,
            pltpu.SemaphoreType.DMA,
            pltpu.SemaphoreType.DMA,
            pltpu.SemaphoreType.DMA,
            pltpu.SemaphoreType.DMA,
            pltpu.SemaphoreType.DMA,
        ],
        compiler_params=(None if tiled else
                         pltpu.CompilerParams(use_tc_tiling_on_sc=False)),
    )
    def agg(h_hbm, src_hbm, dst_hbm, zer_hbm, out_hbm, idxs, idxd,
            ra, rb, rc, rd, acc, g0, g1, g2, g3, s0, s1_, s2_, s3_):
        c = lax.axis_index("c")
        s = lax.axis_index("s")
        w = s * NC + c
        r0 = s * RPT
        pltpu.sync_copy(zer_hbm.at[pl.ds(r0, RPT)], acc.at[pl.ds(r0, RPT)])
        pltpu.sync_copy(src_hbm.at[pl.ds(w * NWIN, NWIN)], idxs)
        plsc.subcore_barrier()
        for half in range(2):
            pltpu.sync_copy(
                dst_hbm.at[pl.ds(w * NWIN + half * NHALF, NHALF)], idxd)
            _edge_loop_q(h_hbm, idxs, half * NHALF, (ra, rb, rc, rd), acc,
                         (g0, g1, g2, g3), (s0, s1_, s2_, s3_),
                         NHALF, lambda j: idxd.at[j])
        plsc.subcore_barrier()
        pltpu.sync_copy(acc.at[pl.ds(r0, RPT)],
                        out_hbm.at[c, pl.ds(r0, RPT)])

    return agg


WL1 = 64
BF16 = jnp.bfloat16
_agg_l1 = _make_edge_split_agg(C1, WL1, BF16)
_agg_l3 = _make_edge_split_agg(DOUT, WL1, jnp.float32, tiled=True)


C2 = 256


def _make_chunk_agg():
    W2 = 64
    EPT = EPAD // NS
    NWIN = EPT // W2

    NHALF = NWIN // 2

    @functools.partial(
        pl.kernel,
        out_type=[jax.ShapeDtypeStruct((NPAD, C2), BF16)] * 2,
        mesh=_mesh,
        scratch_types=[
            pltpu.VMEM((NWIN, W2), jnp.int32),
            pltpu.VMEM((NHALF, W2), jnp.int32),
            pltpu.VMEM((W2, C2), BF16),
            pltpu.VMEM((W2, C2), BF16),
            pltpu.VMEM((W2, C2), BF16),
            pltpu.VMEM((W2, C2), BF16),
            pltpu.VMEM_SHARED((NPAD, C2), BF16),
            pltpu.SemaphoreType.DMA,
            pltpu.SemaphoreType.DMA,
            pltpu.SemaphoreType.DMA,
            pltpu.SemaphoreType.DMA,
            pltpu.SemaphoreType.DMA,
            pltpu.SemaphoreType.DMA,
            pltpu.SemaphoreType.DMA,
            pltpu.SemaphoreType.DMA,
        ],
        compiler_params=pltpu.CompilerParams(use_tc_tiling_on_sc=False),
    )
    def agg2(h0, h1, src_hbm, dst_hbm, zer_hbm,
             o0, o1, idxs, idxd, ra, rb, rc, rd,
             acc, g0, g1, g2, g3, s0, s1_, s2_, s3_):
        c = lax.axis_index("c")
        s = lax.axis_index("s")
        r0 = s * RPT
        hs = (h0, h1)
        os_ = (o0, o1)
        rows = (ra, rb, rc, rd)
        gs = (g0, g1, g2, g3)
        ss = (s0, s1_, s2_, s3_)
        pltpu.sync_copy(src_hbm.at[pl.ds(s * NWIN, NWIN)], idxs)

        for chunk in range(2):
            h_hbm = hs[chunk]
            out_hbm = os_[chunk]

            @pl.when(c == chunk)
            def _process():
                pltpu.sync_copy(zer_hbm.at[pl.ds(r0, RPT)],
                                acc.at[pl.ds(r0, RPT)])
                plsc.subcore_barrier()
                for half in range(2):
                    pltpu.sync_copy(
                        dst_hbm.at[pl.ds(s * NWIN + half * NHALF, NHALF)],
                        idxd)
                    _edge_loop_q(h_hbm, idxs, half * NHALF, rows,
                                 acc, gs, ss, NHALF, lambda j: idxd.at[j])
                plsc.subcore_barrier()
                pltpu.sync_copy(acc.at[pl.ds(r0, RPT)],
                                out_hbm.at[pl.ds(r0, RPT)])

    return agg2


_agg_l2 = _make_chunk_agg()

R = 256
NBLK = NPAD // R


def _l1_body(s1_ref, x_ref, wl_ref, bl_ref, wr_ref,
             h0_ref, h1_ref, rb_ref):
    ssum = s1_ref[0] + s1_ref[1]
    cnt = ssum[:, DIN:DIN + 1].astype(jnp.float32)
    recip = 1.0 / jnp.maximum(cnt, 1.0)
    h = (jnp.dot(ssum[:, :DIN], wl_ref[...].astype(jnp.bfloat16),
                 preferred_element_type=jnp.float32) * recip
         + bl_ref[...]
         + jnp.dot(x_ref[...], wr_ref[...].astype(jnp.bfloat16),
                   preferred_element_type=jnp.float32))
    h = jnp.maximum(h, 0.0)
    hb = h.astype(jnp.bfloat16)
    h0_ref[...] = hb[:, 0:C2]
    h1_ref[...] = hb[:, C2:DH]
    rb_ref[...] = jnp.broadcast_to(recip, (R, DIN)).astype(jnp.bfloat16)


def _tc_layer1(s1, x_pad, wl1, bl1, wr1):
    blk = lambda i: (i, 0)
    whole = lambda i: (0, 0)
    outs_bf = jax.ShapeDtypeStruct((NPAD, C2), BF16)
    outs_rb = jax.ShapeDtypeStruct((NPAD, DIN), BF16)
    return pl.pallas_call(
        _l1_body,
        grid=(NBLK,),
        in_specs=[
            pl.BlockSpec((2, R, C1), lambda i: (0, i, 0)),
            pl.BlockSpec((R, DIN), blk),
            pl.BlockSpec((DIN, DH), whole),
            pl.BlockSpec((1, DH), whole),
            pl.BlockSpec((DIN, DH), whole),
        ],
        out_specs=[pl.BlockSpec((R, C2), blk)] * 2
        + [pl.BlockSpec((R, DIN), blk)],
        out_shape=[outs_bf] * 2 + [outs_rb],
    )(s1, x_pad, wl1, bl1, wr1)


def _l2_body(s20, s21, h10, h11, rb_ref,
             wl2_ref, bl2_ref, wr2_ref, wl3_ref, wr3_ref,
             p3_ref, r3_ref):
    recip = rb_ref[:, 0:1].astype(jnp.float32)
    aggv = jnp.concatenate([s20[...], s21[...]], axis=1)
    h1 = jnp.concatenate([h10[...], h11[...]], axis=1)
    h2 = (jnp.dot(aggv, wl2_ref[...].astype(jnp.bfloat16),
                  preferred_element_type=jnp.float32) * recip
          + bl2_ref[...]
          + jnp.dot(h1, wr2_ref[...].astype(jnp.bfloat16),
                    preferred_element_type=jnp.float32))
    h2 = jnp.maximum(h2, 0.0).astype(jnp.bfloat16)
    p3_ref[...] = jnp.dot(h2, wl3_ref[...].astype(jnp.bfloat16),
                          preferred_element_type=jnp.float32)
    r3_ref[...] = jnp.dot(h2, wr3_ref[...].astype(jnp.bfloat16),
                          preferred_element_type=jnp.float32)


def _tc_layer2(s2s, h1s, recipb, wl2, bl2, wr2, wl3, wr3):
    blk = lambda i: (i, 0)
    whole = lambda i: (0, 0)
    outs_f32 = jax.ShapeDtypeStruct((NPAD, DOUT), jnp.float32)
    return pl.pallas_call(
        _l2_body,
        grid=(NBLK,),
        in_specs=(
            [pl.BlockSpec((R, C2), blk)] * 4
            + [pl.BlockSpec((R, DIN), blk)]
            + [pl.BlockSpec((DH, DH), whole),
               pl.BlockSpec((1, DH), whole),
               pl.BlockSpec((DH, DH), whole),
               pl.BlockSpec((DH, DOUT), whole),
               pl.BlockSpec((DH, DOUT), whole)]
        ),
        out_specs=[pl.BlockSpec((R, DOUT), blk)] * 2,
        out_shape=[outs_f32, outs_f32],
    )(*s2s, *h1s, recipb, wl2, bl2, wr2, wl3, wr3)


def _final_body(s3_ref, rb_ref, r3_ref, b_ref, bl3_ref, g_ref, be_ref,
                out_ref, psum, csum):
    i = pl.program_id(0)

    @pl.when(i == 0)
    def _init():
        psum[...] = jnp.zeros((G, DOUT), jnp.float32)
        csum[...] = jnp.zeros((G, 1), jnp.float32)

    ssum = s3_ref[0].astype(jnp.float32) + s3_ref[1].astype(jnp.float32)
    out3 = (ssum * rb_ref[:, 0:1].astype(jnp.float32)
            + r3_ref[...] + bl3_ref[...])
    bb = b_ref[0]
    gids = lax.broadcasted_iota(jnp.int32, (G, R), 0).astype(jnp.float32)
    onehot = jnp.where(gids == bb, 1.0, 0.0)
    psum[...] += jnp.dot(onehot, out3, preferred_element_type=jnp.float32)
    csum[...] += jnp.sum(onehot, axis=1, keepdims=True)

    @pl.when(i == NBLK - 1)
    def _finish():
        pooled = psum[...] / jnp.maximum(csum[...], 1.0)
        mu = jnp.mean(pooled, axis=1, keepdims=True)
        var = jnp.mean((pooled - mu) ** 2, axis=1, keepdims=True)
        out_ref[...] = ((pooled - mu) * lax.rsqrt(var + 1e-5)
                        * g_ref[...] + be_ref[...])


def _tc_final(s3, recipb, r3, batchf, bl3, ln_g, ln_b):
    blk = lambda i: (i, 0)
    whole = lambda i: (0, 0)
    return pl.pallas_call(
        _final_body,
        grid=(NBLK,),
        in_specs=[
            pl.BlockSpec((2, R, DOUT), lambda i: (0, i, 0)),
            pl.BlockSpec((R, DIN), blk),
            pl.BlockSpec((R, DOUT), blk),
            pl.BlockSpec((1, 1, R), lambda i: (i, 0, 0)),
            pl.BlockSpec((1, DOUT), whole),
            pl.BlockSpec((1, DOUT), whole),
            pl.BlockSpec((1, DOUT), whole),
        ],
        out_specs=pl.BlockSpec((G, DOUT), whole),
        out_shape=jax.ShapeDtypeStruct((G, DOUT), jnp.float32),
        scratch_shapes=[
            pltpu.VMEM((G, DOUT), jnp.float32),
            pltpu.VMEM((G, 1), jnp.float32),
        ],
    )(s3, recipb, r3, batchf, bl3, ln_g, ln_b)


def kernel(x, edge_index, batch, Wl1, bl1, Wr1, Wl2, bl2, Wr2,
           Wl3, bl3, Wr3, ln_g, ln_b):
    f32 = jnp.float32
    src = edge_index[0]
    dst = edge_index[1]
    padidx = (N + (jnp.arange(EPAD - E, dtype=jnp.int32) % (NPAD - N)))
    srcf = jnp.concatenate([src, padidx])
    dstf = jnp.concatenate([dst, padidx])
    srcp64 = srcf.reshape(EPAD // 64, 64)
    dstp64 = dstf.reshape(EPAD // 64, 64)

    xa = jnp.concatenate(
        [x, jnp.ones((N, 1), f32), jnp.zeros((N, C1 - DIN - 1), f32)], axis=1)
    xa = jnp.concatenate([xa, jnp.zeros((NPAD - N, C1), f32)], axis=0)
    xa = xa.astype(BF16)
    x_pad = jnp.concatenate([x, jnp.zeros((NPAD - N, DIN), f32)],
                            axis=0).astype(BF16)

    zer1 = jnp.zeros((NPAD, C1), BF16)
    zer2 = jnp.zeros((NPAD, C2), BF16)
    zer3 = jnp.zeros((NPAD, DIN), f32)

    s1 = _agg_l1(xa, srcp64, dstp64, zer1)
    h1s_and_recip = _tc_layer1(s1, x_pad, Wl1, bl1.reshape(1, DH), Wr1)
    h1s, recipb = h1s_and_recip[:2], h1s_and_recip[2]

    s2s = _agg_l2(*h1s, srcp64, dstp64, zer2)
    p3, r3 = _tc_layer2(s2s, h1s, recipb, Wl2, bl2.reshape(1, DH), Wr2,
                        Wl3, Wr3)

    s3 = _agg_l3(p3, srcp64, dstp64, zer3)
    batchf = jnp.concatenate(
        [batch.astype(f32), jnp.full((NPAD - N,), float(G), f32)]
    ).reshape(NBLK, 1, R)
    out = _tc_final(s3, recipb, r3, batchf, bl3.reshape(1, DOUT),
                    ln_g.reshape(1, DOUT), ln_b.reshape(1, DOUT))
    return out
